# Initial kernel scaffold; baseline (speedup 1.0000x reference)
import jax
import jax.numpy as jnp
from jax.experimental import pallas as pl


def kernel(x, edge_index, batch, c1_wr, c1_wro, c1_b, c2_wr, c2_wro, c2_b, c3_wr, c3_wro, c3_b, c4_wr, c4_wro, c4_b, p0_w, p1_w, l1_w, l1_b, l2_w, l2_b):
    raise NotImplementedError("write your pallas kernel here")


# speedup vs baseline: 5.8369x; 5.8369x over previous
import functools

import jax
import jax.numpy as jnp
from jax import lax
from jax.experimental import pallas as pl
from jax.experimental.pallas import tpu as pltpu
from jax.experimental.pallas import tpu_sc as plsc

F32 = jnp.float32
BN = 256
BNJ = 1024
NG = 128
HI = jax.lax.Precision.HIGHEST


def _edge_rowagg(table, src, dst, zeros):
    npad, w = table.shape
    e = src.shape[0]
    nc, ns, b = 2, 16, 80
    ec, es = e // nc, e // (nc * ns)
    nblk = es // b
    rps = npad // ns
    assert es % b == 0 and npad % ns == 0 and rps % 8 == 0

    mesh = plsc.VectorSubcoreMesh(core_axis_name="c", subcore_axis_name="s")

    @functools.partial(
        pl.kernel,
        out_type=jax.ShapeDtypeStruct((nc * npad, w), F32),
        mesh=mesh,
        scratch_types=[
            pltpu.VMEM_SHARED((npad, w), F32),
            pltpu.VMEM((b,), jnp.int32),
            pltpu.VMEM((b,), jnp.int32),
            pltpu.VMEM((b, w), F32),
            pltpu.SemaphoreType.DMA,
        ],
        compiler_params=pltpu.CompilerParams(use_tc_tiling_on_sc=False),
    )
    def agg(tab_hbm, src_hbm, dst_hbm, z_hbm, out_hbm, acc, src_v, dst_v,
            rows_v, sem):
        c = lax.axis_index("c")
        s = lax.axis_index("s")
        r0 = pl.multiple_of(s * rps, 8)
        pltpu.sync_copy(z_hbm.at[pl.ds(r0, rps)], acc.at[pl.ds(r0, rps)])
        plsc.subcore_barrier()
        base = c * ec + s * es

        def body(i, carry):
            off = pl.multiple_of(base + i * b, 8)
            pltpu.sync_copy(src_hbm.at[pl.ds(off, b)], src_v)
            pltpu.async_copy(tab_hbm.at[src_v], rows_v, sem).wait()
            pltpu.sync_copy(dst_hbm.at[pl.ds(off, b)], dst_v)
            pltpu.sync_copy(rows_v, acc.at[dst_v], add=True)
            return carry

        lax.fori_loop(0, nblk, body, 0)
        plsc.subcore_barrier()
        o0 = pl.multiple_of(c * npad + s * rps, 8)
        pltpu.sync_copy(acc.at[pl.ds(r0, rps)], out_hbm.at[pl.ds(o0, rps)])

    return agg(table, src, dst, zeros).reshape(nc, npad, w)


def _tc_pre(xp, w1, w2, bias):
    npad, f = xp.shape
    h = w1.shape[1]
    nb = npad // BN

    def body(x_ref, w1_ref, w2_ref, b_ref, y_ref, r_ref):
        xb = x_ref[...]
        y_ref[...] = jnp.dot(xb, w1_ref[...], precision=HI,
                             preferred_element_type=F32)
        r_ref[...] = jnp.dot(xb, w2_ref[...], precision=HI,
                             preferred_element_type=F32) + b_ref[...]

    return pl.pallas_call(
        body,
        grid=(nb,),
        in_specs=[
            pl.BlockSpec((BN, f), lambda i: (i, 0)),
            pl.BlockSpec((f, h), lambda i: (0, 0)),
            pl.BlockSpec((f, h), lambda i: (0, 0)),
            pl.BlockSpec((1, h), lambda i: (0, 0)),
        ],
        out_specs=[pl.BlockSpec((BN, h), lambda i: (i, 0))] * 2,
        out_shape=[jax.ShapeDtypeStruct((npad, h), F32)] * 2,
    )(xp, w1, w2, bias)


def _combine_h(sp_ref, dp_ref, r_ref, nm):
    s = sp_ref[0] + sp_ref[1]
    d = dp_ref[0][:, 0:1] + dp_ref[1][:, 0:1]
    if nm is None:
        return jnp.maximum(s / jnp.maximum(d, 1.0) + r_ref[...], 0.0)
    return jnp.maximum(s * nm / jnp.maximum(d * nm, 1.0) + r_ref[...],
                       0.0) * nm


def _onehot(br):
    gcol = lax.broadcasted_iota(jnp.int32, (NG, 1), 0).astype(F32)
    return jnp.where(gcol == br, 1.0, 0.0)


def _tc_combine1(sp, dp, r, b_row, w1n, w2n, bn):
    npad, h = r.shape
    nb = npad // BN

    def body(sp_ref, dp_ref, r_ref, br_ref, w1_ref, w2_ref, bn_ref,
             y_ref, r2_ref, xs_ref, cnt_ref):
        i = pl.program_id(0)
        hb = _combine_h(sp_ref, dp_ref, r_ref, None)
        y_ref[...] = jnp.dot(hb, w1_ref[...], precision=HI,
                             preferred_element_type=F32)
        r2_ref[...] = jnp.dot(hb, w2_ref[...], precision=HI,
                              preferred_element_type=F32) + bn_ref[...]
        og = _onehot(br_ref[...])

        @pl.when(i == 0)
        def _():
            xs_ref[...] = jnp.zeros_like(xs_ref)
            cnt_ref[...] = jnp.zeros_like(cnt_ref)

        xs_ref[...] += jnp.dot(og, hb, precision=HI,
                               preferred_element_type=F32)
        cnt_ref[...] += jnp.sum(og, axis=1, keepdims=True)

    return pl.pallas_call(
        body,
        grid=(nb,),
        in_specs=[
            pl.BlockSpec((2, BN, h), lambda i: (0, i, 0)),
            pl.BlockSpec((2, BN, 16), lambda i: (0, i, 0)),
            pl.BlockSpec((BN, h), lambda i: (i, 0)),
            pl.BlockSpec((1, BN), lambda i: (0, i)),
            pl.BlockSpec((h, h), lambda i: (0, 0)),
            pl.BlockSpec((h, h), lambda i: (0, 0)),
            pl.BlockSpec((1, h), lambda i: (0, 0)),
        ],
        out_specs=[
            pl.BlockSpec((BN, h), lambda i: (i, 0)),
            pl.BlockSpec((BN, h), lambda i: (i, 0)),
            pl.BlockSpec((NG, h), lambda i: (0, 0)),
            pl.BlockSpec((NG, 1), lambda i: (0, 0)),
        ],
        out_shape=[
            jax.ShapeDtypeStruct((npad, h), F32),
            jax.ShapeDtypeStruct((npad, h), F32),
            jax.ShapeDtypeStruct((NG, h), F32),
            jax.ShapeDtypeStruct((NG, 1), F32),
        ],
    )(sp, dp, r, b_row, w1n, w2n, bn)


def _tc_combine2(sp, dp, r, b_row, pw_row):
    npad, h = r.shape
    nb = npad // BN

    def body(sp_ref, dp_ref, r_ref, br_ref, pw_ref, h_ref, z_ref, xs_ref):
        i = pl.program_id(0)
        hb = _combine_h(sp_ref, dp_ref, r_ref, None)
        h_ref[...] = hb
        pw = pw_ref[...]
        nrm = jnp.sqrt(jnp.sum(pw * pw))
        z_ref[...] = jnp.sum(hb * pw, axis=1, keepdims=True) / nrm
        og = _onehot(br_ref[...])

        @pl.when(i == 0)
        def _():
            xs_ref[...] = jnp.zeros_like(xs_ref)

        xs_ref[...] += jnp.dot(og, hb, precision=HI,
                               preferred_element_type=F32)

    return pl.pallas_call(
        body,
        grid=(nb,),
        in_specs=[
            pl.BlockSpec((2, BN, h), lambda i: (0, i, 0)),
            pl.BlockSpec((2, BN, 16), lambda i: (0, i, 0)),
            pl.BlockSpec((BN, h), lambda i: (i, 0)),
            pl.BlockSpec((1, BN), lambda i: (0, i)),
            pl.BlockSpec((1, h), lambda i: (0, 0)),
        ],
        out_specs=[
            pl.BlockSpec((BN, h), lambda i: (i, 0)),
            pl.BlockSpec((BN, 1), lambda i: (i, 0)),
            pl.BlockSpec((NG, h), lambda i: (0, 0)),
        ],
        out_shape=[
            jax.ShapeDtypeStruct((npad, h), F32),
            jax.ShapeDtypeStruct((npad, 1), F32),
            jax.ShapeDtypeStruct((NG, h), F32),
        ],
    )(sp, dp, r, b_row, pw_row)


def _tc_rank(z_col, z_row, b_col, b_row):
    npad = z_col.shape[0]
    nbi, nbj = npad // BN, npad // BNJ

    def body(zi_ref, bi_ref, zj_ref, bj_ref, rk_ref):
        i = pl.program_id(0)
        j = pl.program_id(1)

        @pl.when(j == 0)
        def _():
            rk_ref[...] = jnp.zeros_like(rk_ref)

        bi = bi_ref[...]
        bj = bj_ref[...]
        overlap = jnp.logical_and(bj[0, 0] <= bi[BN - 1, 0],
                                  bj[0, BNJ - 1] >= bi[0, 0])

        @pl.when(overlap)
        def _():
            zi = zi_ref[...]
            zj = zj_ref[...]
            beq = bi == bj
            gi = (i * BN + lax.broadcasted_iota(jnp.int32, (BN, 1), 0)
                  ).astype(F32)
            gj = (j * BNJ + lax.broadcasted_iota(jnp.int32, (1, BNJ), 1)
                  ).astype(F32)
            ahead = (zj > zi) | ((zj == zi) & (gj < gi))
            cmat = jnp.where(beq & ahead, 1.0, 0.0)
            rk_ref[...] += jnp.sum(cmat, axis=1, keepdims=True)

    return pl.pallas_call(
        body,
        grid=(nbi, nbj),
        in_specs=[
            pl.BlockSpec((BN, 1), lambda i, j: (i, 0)),
            pl.BlockSpec((BN, 1), lambda i, j: (i, 0)),
            pl.BlockSpec((1, BNJ), lambda i, j: (0, j)),
            pl.BlockSpec((1, BNJ), lambda i, j: (0, j)),
        ],
        out_specs=pl.BlockSpec((BN, 1), lambda i, j: (i, 0)),
        out_shape=jax.ShapeDtypeStruct((npad, 1), F32),
    )(z_col, b_col, z_row, b_row)


def _tc_topk(h2, z, rank, cnt_row, b_col, w1n, w2n, bn):
    npad, h = h2.shape
    nb = npad // BN

    def body(h_ref, z_ref, rk_ref, cnt_ref, bc_ref, w1_ref, w2_ref, bn_ref,
             y_ref, r2_ref, nm_ref):
        grow = lax.broadcasted_iota(jnp.int32, (1, NG), 1).astype(F32)
        ogt = jnp.where(bc_ref[...] == grow, 1.0, 0.0)
        k_row = jnp.ceil(0.8 * cnt_ref[...])
        k_node = jnp.sum(ogt * k_row, axis=1, keepdims=True)
        nm = jnp.where(rk_ref[...] < k_node, 1.0, 0.0)
        x3 = h_ref[...] * (jnp.tanh(z_ref[...]) * nm)
        y_ref[...] = jnp.dot(x3, w1_ref[...], precision=HI,
                             preferred_element_type=F32)
        r2_ref[...] = jnp.dot(x3, w2_ref[...], precision=HI,
                              preferred_element_type=F32) + bn_ref[...]
        nm_ref[...] = nm

    return pl.pallas_call(
        body,
        grid=(nb,),
        in_specs=[
            pl.BlockSpec((BN, h), lambda i: (i, 0)),
            pl.BlockSpec((BN, 1), lambda i: (i, 0)),
            pl.BlockSpec((BN, 1), lambda i: (i, 0)),
            pl.BlockSpec((1, NG), lambda i: (0, 0)),
            pl.BlockSpec((BN, 1), lambda i: (i, 0)),
            pl.BlockSpec((h, h), lambda i: (0, 0)),
            pl.BlockSpec((h, h), lambda i: (0, 0)),
            pl.BlockSpec((1, h), lambda i: (0, 0)),
        ],
        out_specs=[
            pl.BlockSpec((BN, h), lambda i: (i, 0)),
            pl.BlockSpec((BN, h), lambda i: (i, 0)),
            pl.BlockSpec((BN, 1), lambda i: (i, 0)),
        ],
        out_shape=[
            jax.ShapeDtypeStruct((npad, h), F32),
            jax.ShapeDtypeStruct((npad, h), F32),
            jax.ShapeDtypeStruct((npad, 1), F32),
        ],
    )(h2, z, rank, cnt_row, b_col, w1n, w2n, bn)


def _tc_combine3(sp, dp, r, nmask, b_row, w1n, w2n, bn):
    npad, h = r.shape
    nb = npad // BN

    def body(sp_ref, dp_ref, r_ref, nm_ref, br_ref, w1_ref, w2_ref, bn_ref,
             y_ref, r2_ref, xs_ref, cnt_ref):
        i = pl.program_id(0)
        nm = nm_ref[...]
        hb = _combine_h(sp_ref, dp_ref, r_ref, nm)
        y_ref[...] = jnp.dot(hb, w1_ref[...], precision=HI,
                             preferred_element_type=F32)
        r2_ref[...] = jnp.dot(hb, w2_ref[...], precision=HI,
                              preferred_element_type=F32) + bn_ref[...]
        og = _onehot(br_ref[...])

        @pl.when(i == 0)
        def _():
            xs_ref[...] = jnp.zeros_like(xs_ref)
            cnt_ref[...] = jnp.zeros_like(cnt_ref)

        xs_ref[...] += jnp.dot(og, hb, precision=HI,
                               preferred_element_type=F32)
        cnt_ref[...] += jnp.dot(og, nm, precision=HI,
                                preferred_element_type=F32)

    return pl.pallas_call(
        body,
        grid=(nb,),
        in_specs=[
            pl.BlockSpec((2, BN, h), lambda i: (0, i, 0)),
            pl.BlockSpec((2, BN, 16), lambda i: (0, i, 0)),
            pl.BlockSpec((BN, h), lambda i: (i, 0)),
            pl.BlockSpec((BN, 1), lambda i: (i, 0)),
            pl.BlockSpec((1, BN), lambda i: (0, i)),
            pl.BlockSpec((h, h), lambda i: (0, 0)),
            pl.BlockSpec((h, h), lambda i: (0, 0)),
            pl.BlockSpec((1, h), lambda i: (0, 0)),
        ],
        out_specs=[
            pl.BlockSpec((BN, h), lambda i: (i, 0)),
            pl.BlockSpec((BN, h), lambda i: (i, 0)),
            pl.BlockSpec((NG, h), lambda i: (0, 0)),
            pl.BlockSpec((NG, 1), lambda i: (0, 0)),
        ],
        out_shape=[
            jax.ShapeDtypeStruct((npad, h), F32),
            jax.ShapeDtypeStruct((npad, h), F32),
            jax.ShapeDtypeStruct((NG, h), F32),
            jax.ShapeDtypeStruct((NG, 1), F32),
        ],
    )(sp, dp, r, nmask, b_row, w1n, w2n, bn)


def _tc_combine4(sp, dp, r, nmask, b_row):
    npad, h = r.shape
    nb = npad // BN

    def body(sp_ref, dp_ref, r_ref, nm_ref, br_ref, xs_ref):
        i = pl.program_id(0)
        hb = _combine_h(sp_ref, dp_ref, r_ref, nm_ref[...])
        og = _onehot(br_ref[...])

        @pl.when(i == 0)
        def _():
            xs_ref[...] = jnp.zeros_like(xs_ref)

        xs_ref[...] += jnp.dot(og, hb, precision=HI,
                               preferred_element_type=F32)

    return pl.pallas_call(
        body,
        grid=(nb,),
        in_specs=[
            pl.BlockSpec((2, BN, h), lambda i: (0, i, 0)),
            pl.BlockSpec((2, BN, 16), lambda i: (0, i, 0)),
            pl.BlockSpec((BN, h), lambda i: (i, 0)),
            pl.BlockSpec((BN, 1), lambda i: (i, 0)),
            pl.BlockSpec((1, BN), lambda i: (0, i)),
        ],
        out_specs=pl.BlockSpec((NG, h), lambda i: (0, 0)),
        out_shape=jax.ShapeDtypeStruct((NG, h), F32),
    )(sp, dp, r, nmask, b_row)


def _tc_head(xs0, xs1, xs2, xs3, cnt0, cnt1, l1w, l1b, l2wp, l2bp):
    def body(a0, a1, a2, a3, c0, c1, w1, b1, w2, b2, o_ref):
        c0v = jnp.maximum(c0[...], 1.0)
        c1v = jnp.maximum(c1[...], 1.0)
        hcat = jnp.concatenate(
            [a0[...] / c0v, a1[...] / c0v, a2[...] / c1v, a3[...] / c1v],
            axis=1)
        t = jnp.maximum(jnp.dot(hcat, w1[...], precision=HI,
                                preferred_element_type=F32) + b1[...], 0.0)
        lg = jnp.dot(t, w2[...], precision=HI,
                     preferred_element_type=F32) + b2[...]
        colv = lax.broadcasted_iota(jnp.int32, (NG, 128), 1) < 16
        lgm = jnp.where(colv, lg, -1e30)
        m = jnp.max(lgm, axis=1, keepdims=True)
        p = jnp.where(colv, jnp.exp(lgm - m), 0.0)
        lse = jnp.log(jnp.sum(p, axis=1, keepdims=True))
        o_ref[...] = lgm - m - lse

    return pl.pallas_call(
        body,
        out_shape=jax.ShapeDtypeStruct((NG, 128), F32),
    )(xs0, xs1, xs2, xs3, cnt0, cnt1, l1w, l1b, l2wp, l2bp)


def kernel(x, edge_index, batch, c1_wr, c1_wro, c1_b, c2_wr, c2_wro, c2_b,
           c3_wr, c3_wro, c3_b, c4_wr, c4_wro, c4_b, p0_w, p1_w,
           l1_w, l1_b, l2_w, l2_b):
    n, f = x.shape
    h = c1_wr.shape[1]
    npad = ((n + BNJ - 1) // BNJ) * BNJ
    pad = npad - n

    xp = jnp.pad(x, ((0, pad), (0, 0)))
    bf = jnp.pad(batch.astype(F32), (0, pad), constant_values=1e9)
    b_row = bf.reshape(1, npad)
    b_col = bf.reshape(npad, 1)
    src = edge_index[0]
    dst = edge_index[1]
    zeros64 = jnp.zeros((npad, h), F32)
    zeros16 = jnp.zeros((npad, 16), F32)
    ones16 = jnp.ones((npad, 16), F32)

    d0p = _edge_rowagg(ones16, src, dst, zeros16)
    y1, r1 = _tc_pre(xp, c1_wr, c1_wro, c1_b.reshape(1, h))
    s1p = _edge_rowagg(y1, src, dst, zeros64)
    y2, r2, xs0, cnt0 = _tc_combine1(s1p, d0p, r1, b_row, c2_wr, c2_wro,
                                     c2_b.reshape(1, h))
    s2p = _edge_rowagg(y2, src, dst, zeros64)
    h2, z, xs1 = _tc_combine2(s2p, d0p, r2, b_row, p0_w.reshape(1, h))
    rank = _tc_rank(z, z.reshape(1, npad), b_col, b_row)
    y3, r3, nmask = _tc_topk(h2, z, rank, cnt0.reshape(1, NG), b_col,
                             c3_wr, c3_wro, c3_b.reshape(1, h))
    nm16 = jnp.broadcast_to(nmask, (npad, 16))
    d1p = _edge_rowagg(nm16, src, dst, zeros16)
    s3p = _edge_rowagg(y3, src, dst, zeros64)
    y4, r4, xs2, cnt1 = _tc_combine3(s3p, d1p, r3, nmask, b_row,
                                     c4_wr, c4_wro, c4_b.reshape(1, h))
    s4p = _edge_rowagg(y4, src, dst, zeros64)
    xs3 = _tc_combine4(s4p, d1p, r4, nmask, b_row)

    c = l2_w.shape[1]
    l2wp = jnp.pad(l2_w, ((0, 0), (0, 128 - c)))
    l2bp = jnp.pad(l2_b.reshape(1, c), ((0, 0), (0, 128 - c)))
    out = _tc_head(xs0, xs1, xs2, xs3, cnt0, cnt1, l1_w,
                   l1_b.reshape(1, h), l2wp, l2bp)
    return out[:, :c]


# speedup vs baseline: 11.0785x; 1.8980x over previous
import functools

import jax
import jax.numpy as jnp
from jax import lax
from jax.experimental import pallas as pl
from jax.experimental.pallas import tpu as pltpu
from jax.experimental.pallas import tpu_sc as plsc

F32 = jnp.float32
BN = 256
BNJ = 1024
NG = 128
HI = jax.lax.Precision.HIGHEST


def _edge_rowagg(table, src2d, dst2d, zeros):
    npad, w = table.shape
    b = src2d.shape[1]
    e = src2d.shape[0] * b
    nc, ns = 2, 16
    nblk = e // (nc * ns * b)
    rps = npad // ns
    assert e % (nc * ns * b) == 0 and npad % ns == 0 and rps % 8 == 0

    mesh = plsc.VectorSubcoreMesh(core_axis_name="c", subcore_axis_name="s")

    @functools.partial(
        pl.kernel,
        out_type=jax.ShapeDtypeStruct((nc * npad, w), F32),
        mesh=mesh,
        scratch_types=[
            pltpu.VMEM_SHARED((npad, w), F32),
            pltpu.VMEM((nblk, b), jnp.int32),
            pltpu.VMEM((nblk, b), jnp.int32),
            pltpu.VMEM((2, b, w), F32),
            pltpu.SemaphoreType.DMA,
        ],
        compiler_params=pltpu.CompilerParams(use_tc_tiling_on_sc=False),
    )
    def agg(tab_hbm, src_hbm, dst_hbm, z_hbm, out_hbm, acc, src_v, dst_v,
            rows, sem_g):
        c = lax.axis_index("c")
        s = lax.axis_index("s")
        r0 = pl.multiple_of(s * rps, 8)
        pltpu.sync_copy(z_hbm.at[pl.ds(r0, rps)], acc.at[pl.ds(r0, rps)])
        blk0 = (c * ns + s) * nblk
        pltpu.sync_copy(src_hbm.at[pl.ds(blk0, nblk)], src_v)
        pltpu.sync_copy(dst_hbm.at[pl.ds(blk0, nblk)], dst_v)
        plsc.subcore_barrier()

        pltpu.async_copy(tab_hbm.at[src_v.at[0]], rows.at[0], sem_g)

        def body(j, carry):
            jm = lax.rem(j, 2)
            pltpu.make_async_copy(tab_hbm.at[src_v.at[j]], rows.at[jm],
                                  sem_g).wait()

            @pl.when(j + 1 < nblk)
            def _():
                pltpu.async_copy(tab_hbm.at[src_v.at[j + 1]],
                                 rows.at[1 - jm], sem_g)

            pltpu.sync_copy(rows.at[jm], acc.at[dst_v.at[j]], add=True)
            return carry

        lax.fori_loop(0, nblk, body, 0)
        plsc.subcore_barrier()
        o0 = pl.multiple_of(c * npad + s * rps, 8)
        pltpu.sync_copy(acc.at[pl.ds(r0, rps)], out_hbm.at[pl.ds(o0, rps)])

    return agg(table, src2d, dst2d, zeros).reshape(nc, npad, w)


def _tc_pre(xp, w1, w2, bias):
    npad, f = xp.shape
    h = w1.shape[1]
    nb = npad // BN

    def body(x_ref, w1_ref, w2_ref, b_ref, y_ref, r_ref):
        xb = x_ref[...]
        y = jnp.dot(xb, w1_ref[...], precision=HI,
                    preferred_element_type=F32)
        y_ref[...] = jnp.concatenate([y, jnp.ones((BN, 16), F32)], axis=1)
        r_ref[...] = jnp.dot(xb, w2_ref[...], precision=HI,
                             preferred_element_type=F32) + b_ref[...]

    return pl.pallas_call(
        body,
        grid=(nb,),
        in_specs=[
            pl.BlockSpec((BN, f), lambda i: (i, 0)),
            pl.BlockSpec((f, h), lambda i: (0, 0)),
            pl.BlockSpec((f, h), lambda i: (0, 0)),
            pl.BlockSpec((1, h), lambda i: (0, 0)),
        ],
        out_specs=[
            pl.BlockSpec((BN, h + 16), lambda i: (i, 0)),
            pl.BlockSpec((BN, h), lambda i: (i, 0)),
        ],
        out_shape=[
            jax.ShapeDtypeStruct((npad, h + 16), F32),
            jax.ShapeDtypeStruct((npad, h), F32),
        ],
    )(xp, w1, w2, bias)


def _combine_h(sp_ref, dp_ref, r_ref, nm):
    sfull = sp_ref[0] + sp_ref[1]
    if dp_ref is None:
        s = sfull[:, 0:64]
        d = sfull[:, 64:65]
    else:
        s = sfull
        d = dp_ref[0][:, 64:65] + dp_ref[1][:, 64:65]
    if nm is None:
        return jnp.maximum(s / jnp.maximum(d, 1.0) + r_ref[...], 0.0)
    return jnp.maximum(s * nm / jnp.maximum(d * nm, 1.0) + r_ref[...],
                       0.0) * nm


def _onehot(br):
    gcol = lax.broadcasted_iota(jnp.int32, (NG, 1), 0).astype(F32)
    return jnp.where(gcol == br, 1.0, 0.0)


def _tc_combine1(sp, r, b_row, w1n, w2n, bn):
    npad, h = r.shape
    nb = npad // BN

    def body(sp_ref, r_ref, br_ref, w1_ref, w2_ref, bn_ref,
             y_ref, r2_ref, xs_ref, cnt_ref):
        i = pl.program_id(0)
        hb = _combine_h(sp_ref, None, r_ref, None)
        y_ref[...] = jnp.dot(hb, w1_ref[...], precision=HI,
                             preferred_element_type=F32)
        r2_ref[...] = jnp.dot(hb, w2_ref[...], precision=HI,
                              preferred_element_type=F32) + bn_ref[...]
        og = _onehot(br_ref[...])

        @pl.when(i == 0)
        def _():
            xs_ref[...] = jnp.zeros_like(xs_ref)
            cnt_ref[...] = jnp.zeros_like(cnt_ref)

        xs_ref[...] += jnp.dot(og, hb, precision=HI,
                               preferred_element_type=F32)
        cnt_ref[...] += jnp.sum(og, axis=1, keepdims=True)

    return pl.pallas_call(
        body,
        grid=(nb,),
        in_specs=[
            pl.BlockSpec((2, BN, h + 16), lambda i: (0, i, 0)),
            pl.BlockSpec((BN, h), lambda i: (i, 0)),
            pl.BlockSpec((1, BN), lambda i: (0, i)),
            pl.BlockSpec((h, h), lambda i: (0, 0)),
            pl.BlockSpec((h, h), lambda i: (0, 0)),
            pl.BlockSpec((1, h), lambda i: (0, 0)),
        ],
        out_specs=[
            pl.BlockSpec((BN, h), lambda i: (i, 0)),
            pl.BlockSpec((BN, h), lambda i: (i, 0)),
            pl.BlockSpec((NG, h), lambda i: (0, 0)),
            pl.BlockSpec((NG, 1), lambda i: (0, 0)),
        ],
        out_shape=[
            jax.ShapeDtypeStruct((npad, h), F32),
            jax.ShapeDtypeStruct((npad, h), F32),
            jax.ShapeDtypeStruct((NG, h), F32),
            jax.ShapeDtypeStruct((NG, 1), F32),
        ],
    )(sp, r, b_row, w1n, w2n, bn)


def _tc_combine2(sp, dp, r, b_row, pw_row):
    npad, h = r.shape
    nb = npad // BN

    def body(sp_ref, dp_ref, r_ref, br_ref, pw_ref, h_ref, z_ref, xs_ref):
        i = pl.program_id(0)
        hb = _combine_h(sp_ref, dp_ref, r_ref, None)
        h_ref[...] = hb
        pw = pw_ref[...]
        nrm = jnp.sqrt(jnp.sum(pw * pw))
        z_ref[...] = jnp.sum(hb * pw, axis=1, keepdims=True) / nrm
        og = _onehot(br_ref[...])

        @pl.when(i == 0)
        def _():
            xs_ref[...] = jnp.zeros_like(xs_ref)

        xs_ref[...] += jnp.dot(og, hb, precision=HI,
                               preferred_element_type=F32)

    return pl.pallas_call(
        body,
        grid=(nb,),
        in_specs=[
            pl.BlockSpec((2, BN, h), lambda i: (0, i, 0)),
            pl.BlockSpec((2, BN, h + 16), lambda i: (0, i, 0)),
            pl.BlockSpec((BN, h), lambda i: (i, 0)),
            pl.BlockSpec((1, BN), lambda i: (0, i)),
            pl.BlockSpec((1, h), lambda i: (0, 0)),
        ],
        out_specs=[
            pl.BlockSpec((BN, h), lambda i: (i, 0)),
            pl.BlockSpec((BN, 1), lambda i: (i, 0)),
            pl.BlockSpec((NG, h), lambda i: (0, 0)),
        ],
        out_shape=[
            jax.ShapeDtypeStruct((npad, h), F32),
            jax.ShapeDtypeStruct((npad, 1), F32),
            jax.ShapeDtypeStruct((NG, h), F32),
        ],
    )(sp, dp, r, b_row, pw_row)


def _tc_rank(z_col, z_row, b_col, b_row):
    npad = z_col.shape[0]
    nbi, nbj = npad // BN, npad // BNJ

    def body(zi_ref, bi_ref, zj_ref, bj_ref, rk_ref):
        i = pl.program_id(0)
        j = pl.program_id(1)

        @pl.when(j == 0)
        def _():
            rk_ref[...] = jnp.zeros_like(rk_ref)

        bi = bi_ref[...]
        bj = bj_ref[...]
        overlap = jnp.logical_and(bj[0, 0] <= bi[BN - 1, 0],
                                  bj[0, BNJ - 1] >= bi[0, 0])

        @pl.when(overlap)
        def _():
            zi = zi_ref[...]
            zj = zj_ref[...]
            beq = bi == bj
            gi = (i * BN + lax.broadcasted_iota(jnp.int32, (BN, 1), 0)
                  ).astype(F32)
            gj = (j * BNJ + lax.broadcasted_iota(jnp.int32, (1, BNJ), 1)
                  ).astype(F32)
            ahead = (zj > zi) | ((zj == zi) & (gj < gi))
            cmat = jnp.where(beq & ahead, 1.0, 0.0)
            rk_ref[...] += jnp.sum(cmat, axis=1, keepdims=True)

    return pl.pallas_call(
        body,
        grid=(nbi, nbj),
        in_specs=[
            pl.BlockSpec((BN, 1), lambda i, j: (i, 0)),
            pl.BlockSpec((BN, 1), lambda i, j: (i, 0)),
            pl.BlockSpec((1, BNJ), lambda i, j: (0, j)),
            pl.BlockSpec((1, BNJ), lambda i, j: (0, j)),
        ],
        out_specs=pl.BlockSpec((BN, 1), lambda i, j: (i, 0)),
        out_shape=jax.ShapeDtypeStruct((npad, 1), F32),
    )(z_col, b_col, z_row, b_row)


def _tc_topk(h2, z, rank, cnt_row, b_col, w1n, w2n, bn):
    npad, h = h2.shape
    nb = npad // BN

    def body(h_ref, z_ref, rk_ref, cnt_ref, bc_ref, w1_ref, w2_ref, bn_ref,
             y_ref, r2_ref, nm_ref):
        grow = lax.broadcasted_iota(jnp.int32, (1, NG), 1).astype(F32)
        ogt = jnp.where(bc_ref[...] == grow, 1.0, 0.0)
        k_row = jnp.ceil(0.8 * cnt_ref[...])
        k_node = jnp.sum(ogt * k_row, axis=1, keepdims=True)
        nm = jnp.where(rk_ref[...] < k_node, 1.0, 0.0)
        x3 = h_ref[...] * (jnp.tanh(z_ref[...]) * nm)
        y = jnp.dot(x3, w1_ref[...], precision=HI,
                    preferred_element_type=F32)
        y_ref[...] = jnp.concatenate(
            [y, jnp.broadcast_to(nm, (BN, 16))], axis=1)
        r2_ref[...] = jnp.dot(x3, w2_ref[...], precision=HI,
                              preferred_element_type=F32) + bn_ref[...]
        nm_ref[...] = nm

    return pl.pallas_call(
        body,
        grid=(nb,),
        in_specs=[
            pl.BlockSpec((BN, h), lambda i: (i, 0)),
            pl.BlockSpec((BN, 1), lambda i: (i, 0)),
            pl.BlockSpec((BN, 1), lambda i: (i, 0)),
            pl.BlockSpec((1, NG), lambda i: (0, 0)),
            pl.BlockSpec((BN, 1), lambda i: (i, 0)),
            pl.BlockSpec((h, h), lambda i: (0, 0)),
            pl.BlockSpec((h, h), lambda i: (0, 0)),
            pl.BlockSpec((1, h), lambda i: (0, 0)),
        ],
        out_specs=[
            pl.BlockSpec((BN, h + 16), lambda i: (i, 0)),
            pl.BlockSpec((BN, h), lambda i: (i, 0)),
            pl.BlockSpec((BN, 1), lambda i: (i, 0)),
        ],
        out_shape=[
            jax.ShapeDtypeStruct((npad, h + 16), F32),
            jax.ShapeDtypeStruct((npad, h), F32),
            jax.ShapeDtypeStruct((npad, 1), F32),
        ],
    )(h2, z, rank, cnt_row, b_col, w1n, w2n, bn)


def _tc_combine3(sp, r, nmask, b_row, w1n, w2n, bn):
    npad, h = r.shape
    nb = npad // BN

    def body(sp_ref, r_ref, nm_ref, br_ref, w1_ref, w2_ref, bn_ref,
             y_ref, r2_ref, xs_ref, cnt_ref):
        i = pl.program_id(0)
        nm = nm_ref[...]
        hb = _combine_h(sp_ref, None, r_ref, nm)
        y_ref[...] = jnp.dot(hb, w1_ref[...], precision=HI,
                             preferred_element_type=F32)
        r2_ref[...] = jnp.dot(hb, w2_ref[...], precision=HI,
                              preferred_element_type=F32) + bn_ref[...]
        og = _onehot(br_ref[...])

        @pl.when(i == 0)
        def _():
            xs_ref[...] = jnp.zeros_like(xs_ref)
            cnt_ref[...] = jnp.zeros_like(cnt_ref)

        xs_ref[...] += jnp.dot(og, hb, precision=HI,
                               preferred_element_type=F32)
        cnt_ref[...] += jnp.dot(og, nm, precision=HI,
                                preferred_element_type=F32)

    return pl.pallas_call(
        body,
        grid=(nb,),
        in_specs=[
            pl.BlockSpec((2, BN, h + 16), lambda i: (0, i, 0)),
            pl.BlockSpec((BN, h), lambda i: (i, 0)),
            pl.BlockSpec((BN, 1), lambda i: (i, 0)),
            pl.BlockSpec((1, BN), lambda i: (0, i)),
            pl.BlockSpec((h, h), lambda i: (0, 0)),
            pl.BlockSpec((h, h), lambda i: (0, 0)),
            pl.BlockSpec((1, h), lambda i: (0, 0)),
        ],
        out_specs=[
            pl.BlockSpec((BN, h), lambda i: (i, 0)),
            pl.BlockSpec((BN, h), lambda i: (i, 0)),
            pl.BlockSpec((NG, h), lambda i: (0, 0)),
            pl.BlockSpec((NG, 1), lambda i: (0, 0)),
        ],
        out_shape=[
            jax.ShapeDtypeStruct((npad, h), F32),
            jax.ShapeDtypeStruct((npad, h), F32),
            jax.ShapeDtypeStruct((NG, h), F32),
            jax.ShapeDtypeStruct((NG, 1), F32),
        ],
    )(sp, r, nmask, b_row, w1n, w2n, bn)


def _tc_combine4(sp, dp, r, nmask, b_row):
    npad, h = r.shape
    nb = npad // BN

    def body(sp_ref, dp_ref, r_ref, nm_ref, br_ref, xs_ref):
        i = pl.program_id(0)
        hb = _combine_h(sp_ref, dp_ref, r_ref, nm_ref[...])
        og = _onehot(br_ref[...])

        @pl.when(i == 0)
        def _():
            xs_ref[...] = jnp.zeros_like(xs_ref)

        xs_ref[...] += jnp.dot(og, hb, precision=HI,
                               preferred_element_type=F32)

    return pl.pallas_call(
        body,
        grid=(nb,),
        in_specs=[
            pl.BlockSpec((2, BN, h), lambda i: (0, i, 0)),
            pl.BlockSpec((2, BN, h + 16), lambda i: (0, i, 0)),
            pl.BlockSpec((BN, h), lambda i: (i, 0)),
            pl.BlockSpec((BN, 1), lambda i: (i, 0)),
            pl.BlockSpec((1, BN), lambda i: (0, i)),
        ],
        out_specs=pl.BlockSpec((NG, h), lambda i: (0, 0)),
        out_shape=jax.ShapeDtypeStruct((NG, h), F32),
    )(sp, dp, r, nmask, b_row)


def _tc_head(xs0, xs1, xs2, xs3, cnt0, cnt1, l1w, l1b, l2wp, l2bp):
    def body(a0, a1, a2, a3, c0, c1, w1, b1, w2, b2, o_ref):
        c0v = jnp.maximum(c0[...], 1.0)
        c1v = jnp.maximum(c1[...], 1.0)
        hcat = jnp.concatenate(
            [a0[...] / c0v, a1[...] / c0v, a2[...] / c1v, a3[...] / c1v],
            axis=1)
        t = jnp.maximum(jnp.dot(hcat, w1[...], precision=HI,
                                preferred_element_type=F32) + b1[...], 0.0)
        lg = jnp.dot(t, w2[...], precision=HI,
                     preferred_element_type=F32) + b2[...]
        colv = lax.broadcasted_iota(jnp.int32, (NG, 128), 1) < 16
        lgm = jnp.where(colv, lg, -1e30)
        m = jnp.max(lgm, axis=1, keepdims=True)
        p = jnp.where(colv, jnp.exp(lgm - m), 0.0)
        lse = jnp.log(jnp.sum(p, axis=1, keepdims=True))
        o_ref[...] = lgm - m - lse

    return pl.pallas_call(
        body,
        out_shape=jax.ShapeDtypeStruct((NG, 128), F32),
    )(xs0, xs1, xs2, xs3, cnt0, cnt1, l1w, l1b, l2wp, l2bp)


def kernel(x, edge_index, batch, c1_wr, c1_wro, c1_b, c2_wr, c2_wro, c2_b,
           c3_wr, c3_wro, c3_b, c4_wr, c4_wro, c4_b, p0_w, p1_w,
           l1_w, l1_b, l2_w, l2_b):
    n, f = x.shape
    h = c1_wr.shape[1]
    npad = ((n + BNJ - 1) // BNJ) * BNJ
    pad = npad - n

    xp = jnp.pad(x, ((0, pad), (0, 0)))
    bf = jnp.pad(batch.astype(F32), (0, pad), constant_values=1e9)
    b_row = bf.reshape(1, npad)
    b_col = bf.reshape(npad, 1)
    eb = 80
    src2d = edge_index[0].reshape(-1, eb)
    dst2d = edge_index[1].reshape(-1, eb)
    zeros80 = jnp.zeros((npad, h + 16), F32)
    zeros64 = zeros80[:, :h]

    y1, r1 = _tc_pre(xp, c1_wr, c1_wro, c1_b.reshape(1, h))
    s1p = _edge_rowagg(y1, src2d, dst2d, zeros80)
    y2, r2, xs0, cnt0 = _tc_combine1(s1p, r1, b_row, c2_wr, c2_wro,
                                     c2_b.reshape(1, h))
    s2p = _edge_rowagg(y2, src2d, dst2d, zeros64)
    h2, z, xs1 = _tc_combine2(s2p, s1p, r2, b_row, p0_w.reshape(1, h))
    rank = _tc_rank(z, z.reshape(1, npad), b_col, b_row)
    y3, r3, nmask = _tc_topk(h2, z, rank, cnt0.reshape(1, NG), b_col,
                             c3_wr, c3_wro, c3_b.reshape(1, h))
    s3p = _edge_rowagg(y3, src2d, dst2d, zeros80)
    y4, r4, xs2, cnt1 = _tc_combine3(s3p, r3, nmask, b_row,
                                     c4_wr, c4_wro, c4_b.reshape(1, h))
    s4p = _edge_rowagg(y4, src2d, dst2d, zeros64)
    xs3 = _tc_combine4(s4p, s3p, r4, nmask, b_row)

    c = l2_w.shape[1]
    l2wp = jnp.pad(l2_w, ((0, 0), (0, 128 - c)))
    l2bp = jnp.pad(l2_b.reshape(1, c), ((0, 0), (0, 128 - c)))
    out = _tc_head(xs0, xs1, xs2, xs3, cnt0, cnt1, l1_w,
                   l1_b.reshape(1, h), l2wp, l2bp)
    return out[:, :c]


# speedup vs baseline: 12.2318x; 1.1041x over previous
import functools

import jax
import jax.numpy as jnp
from jax import lax
from jax.experimental import pallas as pl
from jax.experimental.pallas import tpu as pltpu
from jax.experimental.pallas import tpu_sc as plsc

F32 = jnp.float32
BN = 256
BNJ = 1024
NG = 128
HI = jax.lax.Precision.HIGHEST


def _edge_rowagg(table, src2d, dst2d, zeros):
    npad, w = table.shape
    b = src2d.shape[1]
    e = src2d.shape[0] * b
    nc, ns = 2, 16
    nblk = e // (nc * ns * b)
    rps = npad // ns
    assert e % (nc * ns * b) == 0 and npad % ns == 0 and rps % 8 == 0

    mesh = plsc.VectorSubcoreMesh(core_axis_name="c", subcore_axis_name="s")

    @functools.partial(
        pl.kernel,
        out_type=jax.ShapeDtypeStruct((nc * npad, w), F32),
        mesh=mesh,
        scratch_types=[
            pltpu.VMEM_SHARED((npad, w), F32),
            pltpu.VMEM((nblk, b), jnp.int32),
            pltpu.VMEM((nblk, b), jnp.int32),
            pltpu.VMEM((2, b, w), F32),
            pltpu.SemaphoreType.DMA,
        ],
        compiler_params=pltpu.CompilerParams(use_tc_tiling_on_sc=False),
    )
    def agg(tab_hbm, src_hbm, dst_hbm, z_hbm, out_hbm, acc, src_v,
            dst_v, rows, sem_g):
        c = lax.axis_index("c")
        s = lax.axis_index("s")
        r0 = pl.multiple_of(s * rps, 8)
        pltpu.sync_copy(z_hbm.at[pl.ds(r0, rps)], acc.at[pl.ds(r0, rps)])
        blk0 = (c * ns + s) * nblk
        pltpu.sync_copy(src_hbm.at[pl.ds(blk0, nblk)], src_v)
        pltpu.sync_copy(dst_hbm.at[pl.ds(blk0, nblk)], dst_v)
        plsc.subcore_barrier()

        pltpu.async_copy(tab_hbm.at[src_v.at[0]], rows.at[0], sem_g)

        def body(j, carry):
            jm = lax.rem(j, 2)
            pltpu.make_async_copy(tab_hbm.at[src_v.at[j]], rows.at[jm],
                                  sem_g).wait()

            @pl.when(j + 1 < nblk)
            def _():
                pltpu.async_copy(tab_hbm.at[src_v.at[j + 1]],
                                 rows.at[1 - jm], sem_g)

            pltpu.sync_copy(rows.at[jm], acc.at[dst_v.at[j]], add=True)
            return carry

        lax.fori_loop(0, nblk, body, 0)
        plsc.subcore_barrier()
        o0 = pl.multiple_of(c * npad + s * rps, 8)
        pltpu.sync_copy(acc.at[pl.ds(r0, rps)], out_hbm.at[pl.ds(o0, rps)])

    return agg(table, src2d, dst2d, zeros).reshape(nc, npad, w)


def _tc_pre(xp, w1, w2, bias):
    npad, f = xp.shape
    h = w1.shape[1]
    nb = npad // BN

    def body(x_ref, w1_ref, w2_ref, b_ref, y_ref, r_ref):
        xb = x_ref[...]
        y = jnp.dot(xb, w1_ref[...], precision=HI,
                    preferred_element_type=F32)
        y_ref[...] = jnp.concatenate([y, jnp.ones((BN, 16), F32)], axis=1)
        r_ref[...] = jnp.dot(xb, w2_ref[...], precision=HI,
                             preferred_element_type=F32) + b_ref[...]

    return pl.pallas_call(
        body,
        grid=(nb,),
        in_specs=[
            pl.BlockSpec((BN, f), lambda i: (i, 0)),
            pl.BlockSpec((f, h), lambda i: (0, 0)),
            pl.BlockSpec((f, h), lambda i: (0, 0)),
            pl.BlockSpec((1, h), lambda i: (0, 0)),
        ],
        out_specs=[
            pl.BlockSpec((BN, h + 16), lambda i: (i, 0)),
            pl.BlockSpec((BN, h), lambda i: (i, 0)),
        ],
        out_shape=[
            jax.ShapeDtypeStruct((npad, h + 16), F32),
            jax.ShapeDtypeStruct((npad, h), F32),
        ],
    )(xp, w1, w2, bias)


def _combine_h(sp_ref, dp_ref, r_ref, nm):
    sfull = sp_ref[0] + sp_ref[1]
    if dp_ref is None:
        s = sfull[:, 0:64]
        d = sfull[:, 64:65]
    else:
        s = sfull
        d = dp_ref[0][:, 64:65] + dp_ref[1][:, 64:65]
    if nm is None:
        return jnp.maximum(s / jnp.maximum(d, 1.0) + r_ref[...], 0.0)
    return jnp.maximum(s * nm / jnp.maximum(d * nm, 1.0) + r_ref[...],
                       0.0) * nm


def _onehot(br):
    gcol = lax.broadcasted_iota(jnp.int32, (NG, 1), 0).astype(F32)
    return jnp.where(gcol == br, 1.0, 0.0)


def _tc_combine1(sp, r, b_row, w1n, w2n, bn):
    npad, h = r.shape
    nb = npad // BN

    def body(sp_ref, r_ref, br_ref, w1_ref, w2_ref, bn_ref,
             y_ref, r2_ref, xs_ref, cnt_ref):
        i = pl.program_id(0)
        hb = _combine_h(sp_ref, None, r_ref, None)
        y_ref[...] = jnp.dot(hb, w1_ref[...], precision=HI,
                             preferred_element_type=F32)
        r2_ref[...] = jnp.dot(hb, w2_ref[...], precision=HI,
                              preferred_element_type=F32) + bn_ref[...]
        og = _onehot(br_ref[...])

        @pl.when(i == 0)
        def _():
            xs_ref[...] = jnp.zeros_like(xs_ref)
            cnt_ref[...] = jnp.zeros_like(cnt_ref)

        xs_ref[...] += jnp.dot(og, hb, precision=HI,
                               preferred_element_type=F32)
        cnt_ref[...] += jnp.sum(og, axis=1, keepdims=True)

    return pl.pallas_call(
        body,
        grid=(nb,),
        in_specs=[
            pl.BlockSpec((2, BN, h + 16), lambda i: (0, i, 0)),
            pl.BlockSpec((BN, h), lambda i: (i, 0)),
            pl.BlockSpec((1, BN), lambda i: (0, i)),
            pl.BlockSpec((h, h), lambda i: (0, 0)),
            pl.BlockSpec((h, h), lambda i: (0, 0)),
            pl.BlockSpec((1, h), lambda i: (0, 0)),
        ],
        out_specs=[
            pl.BlockSpec((BN, h), lambda i: (i, 0)),
            pl.BlockSpec((BN, h), lambda i: (i, 0)),
            pl.BlockSpec((NG, h), lambda i: (0, 0)),
            pl.BlockSpec((NG, 1), lambda i: (0, 0)),
        ],
        out_shape=[
            jax.ShapeDtypeStruct((npad, h), F32),
            jax.ShapeDtypeStruct((npad, h), F32),
            jax.ShapeDtypeStruct((NG, h), F32),
            jax.ShapeDtypeStruct((NG, 1), F32),
        ],
    )(sp, r, b_row, w1n, w2n, bn)


def _tc_combine2(sp, dp, r, b_row, pw_row):
    npad, h = r.shape
    nb = npad // BN

    def body(sp_ref, dp_ref, r_ref, br_ref, pw_ref, h_ref, z_ref, xs_ref):
        i = pl.program_id(0)
        hb = _combine_h(sp_ref, dp_ref, r_ref, None)
        h_ref[...] = hb
        pw = pw_ref[...]
        nrm = jnp.sqrt(jnp.sum(pw * pw))
        z_ref[...] = jnp.sum(hb * pw, axis=1, keepdims=True) / nrm
        og = _onehot(br_ref[...])

        @pl.when(i == 0)
        def _():
            xs_ref[...] = jnp.zeros_like(xs_ref)

        xs_ref[...] += jnp.dot(og, hb, precision=HI,
                               preferred_element_type=F32)

    return pl.pallas_call(
        body,
        grid=(nb,),
        in_specs=[
            pl.BlockSpec((2, BN, h), lambda i: (0, i, 0)),
            pl.BlockSpec((2, BN, h + 16), lambda i: (0, i, 0)),
            pl.BlockSpec((BN, h), lambda i: (i, 0)),
            pl.BlockSpec((1, BN), lambda i: (0, i)),
            pl.BlockSpec((1, h), lambda i: (0, 0)),
        ],
        out_specs=[
            pl.BlockSpec((BN, h), lambda i: (i, 0)),
            pl.BlockSpec((BN, 1), lambda i: (i, 0)),
            pl.BlockSpec((NG, h), lambda i: (0, 0)),
        ],
        out_shape=[
            jax.ShapeDtypeStruct((npad, h), F32),
            jax.ShapeDtypeStruct((npad, 1), F32),
            jax.ShapeDtypeStruct((NG, h), F32),
        ],
    )(sp, dp, r, b_row, pw_row)


def _tc_rank(z_col, z_row, b_col, b_row):
    npad = z_col.shape[0]
    nbi, nbj = npad // BN, npad // BNJ

    def body(zi_ref, bi_ref, zj_ref, bj_ref, rk_ref):
        i = pl.program_id(0)
        j = pl.program_id(1)

        @pl.when(j == 0)
        def _():
            rk_ref[...] = jnp.zeros_like(rk_ref)

        bi = bi_ref[...]
        bj = bj_ref[...]
        overlap = jnp.logical_and(bj[0, 0] <= bi[BN - 1, 0],
                                  bj[0, BNJ - 1] >= bi[0, 0])

        @pl.when(overlap)
        def _():
            zi = zi_ref[...]
            zj = zj_ref[...]
            beq = bi == bj
            gi = (i * BN + lax.broadcasted_iota(jnp.int32, (BN, 1), 0)
                  ).astype(F32)
            gj = (j * BNJ + lax.broadcasted_iota(jnp.int32, (1, BNJ), 1)
                  ).astype(F32)
            ahead = (zj > zi) | ((zj == zi) & (gj < gi))
            cmat = jnp.where(beq & ahead, 1.0, 0.0)
            rk_ref[...] += jnp.sum(cmat, axis=1, keepdims=True)

    return pl.pallas_call(
        body,
        grid=(nbi, nbj),
        in_specs=[
            pl.BlockSpec((BN, 1), lambda i, j: (i, 0)),
            pl.BlockSpec((BN, 1), lambda i, j: (i, 0)),
            pl.BlockSpec((1, BNJ), lambda i, j: (0, j)),
            pl.BlockSpec((1, BNJ), lambda i, j: (0, j)),
        ],
        out_specs=pl.BlockSpec((BN, 1), lambda i, j: (i, 0)),
        out_shape=jax.ShapeDtypeStruct((npad, 1), F32),
    )(z_col, b_col, z_row, b_row)


def _tc_topk(h2, z, rank, cnt_row, b_col, w1n, w2n, bn):
    npad, h = h2.shape
    nb = npad // BN

    def body(h_ref, z_ref, rk_ref, cnt_ref, bc_ref, w1_ref, w2_ref, bn_ref,
             y_ref, r2_ref, nm_ref):
        grow = lax.broadcasted_iota(jnp.int32, (1, NG), 1).astype(F32)
        ogt = jnp.where(bc_ref[...] == grow, 1.0, 0.0)
        k_row = jnp.ceil(0.8 * cnt_ref[...])
        k_node = jnp.sum(ogt * k_row, axis=1, keepdims=True)
        nm = jnp.where(rk_ref[...] < k_node, 1.0, 0.0)
        x3 = h_ref[...] * (jnp.tanh(z_ref[...]) * nm)
        y = jnp.dot(x3, w1_ref[...], precision=HI,
                    preferred_element_type=F32)
        y_ref[...] = jnp.concatenate(
            [y, jnp.broadcast_to(nm, (BN, 16))], axis=1)
        r2_ref[...] = jnp.dot(x3, w2_ref[...], precision=HI,
                              preferred_element_type=F32) + bn_ref[...]
        nm_ref[...] = nm

    return pl.pallas_call(
        body,
        grid=(nb,),
        in_specs=[
            pl.BlockSpec((BN, h), lambda i: (i, 0)),
            pl.BlockSpec((BN, 1), lambda i: (i, 0)),
            pl.BlockSpec((BN, 1), lambda i: (i, 0)),
            pl.BlockSpec((1, NG), lambda i: (0, 0)),
            pl.BlockSpec((BN, 1), lambda i: (i, 0)),
            pl.BlockSpec((h, h), lambda i: (0, 0)),
            pl.BlockSpec((h, h), lambda i: (0, 0)),
            pl.BlockSpec((1, h), lambda i: (0, 0)),
        ],
        out_specs=[
            pl.BlockSpec((BN, h + 16), lambda i: (i, 0)),
            pl.BlockSpec((BN, h), lambda i: (i, 0)),
            pl.BlockSpec((BN, 1), lambda i: (i, 0)),
        ],
        out_shape=[
            jax.ShapeDtypeStruct((npad, h + 16), F32),
            jax.ShapeDtypeStruct((npad, h), F32),
            jax.ShapeDtypeStruct((npad, 1), F32),
        ],
    )(h2, z, rank, cnt_row, b_col, w1n, w2n, bn)


def _tc_combine3(sp, r, nmask, b_row, w1n, w2n, bn):
    npad, h = r.shape
    nb = npad // BN

    def body(sp_ref, r_ref, nm_ref, br_ref, w1_ref, w2_ref, bn_ref,
             y_ref, r2_ref, xs_ref, cnt_ref):
        i = pl.program_id(0)
        nm = nm_ref[...]
        hb = _combine_h(sp_ref, None, r_ref, nm)
        y_ref[...] = jnp.dot(hb, w1_ref[...], precision=HI,
                             preferred_element_type=F32)
        r2_ref[...] = jnp.dot(hb, w2_ref[...], precision=HI,
                              preferred_element_type=F32) + bn_ref[...]
        og = _onehot(br_ref[...])

        @pl.when(i == 0)
        def _():
            xs_ref[...] = jnp.zeros_like(xs_ref)
            cnt_ref[...] = jnp.zeros_like(cnt_ref)

        xs_ref[...] += jnp.dot(og, hb, precision=HI,
                               preferred_element_type=F32)
        cnt_ref[...] += jnp.dot(og, nm, precision=HI,
                                preferred_element_type=F32)

    return pl.pallas_call(
        body,
        grid=(nb,),
        in_specs=[
            pl.BlockSpec((2, BN, h + 16), lambda i: (0, i, 0)),
            pl.BlockSpec((BN, h), lambda i: (i, 0)),
            pl.BlockSpec((BN, 1), lambda i: (i, 0)),
            pl.BlockSpec((1, BN), lambda i: (0, i)),
            pl.BlockSpec((h, h), lambda i: (0, 0)),
            pl.BlockSpec((h, h), lambda i: (0, 0)),
            pl.BlockSpec((1, h), lambda i: (0, 0)),
        ],
        out_specs=[
            pl.BlockSpec((BN, h), lambda i: (i, 0)),
            pl.BlockSpec((BN, h), lambda i: (i, 0)),
            pl.BlockSpec((NG, h), lambda i: (0, 0)),
            pl.BlockSpec((NG, 1), lambda i: (0, 0)),
        ],
        out_shape=[
            jax.ShapeDtypeStruct((npad, h), F32),
            jax.ShapeDtypeStruct((npad, h), F32),
            jax.ShapeDtypeStruct((NG, h), F32),
            jax.ShapeDtypeStruct((NG, 1), F32),
        ],
    )(sp, r, nmask, b_row, w1n, w2n, bn)


def _tc_combine4(sp, dp, r, nmask, b_row):
    npad, h = r.shape
    nb = npad // BN

    def body(sp_ref, dp_ref, r_ref, nm_ref, br_ref, xs_ref):
        i = pl.program_id(0)
        hb = _combine_h(sp_ref, dp_ref, r_ref, nm_ref[...])
        og = _onehot(br_ref[...])

        @pl.when(i == 0)
        def _():
            xs_ref[...] = jnp.zeros_like(xs_ref)

        xs_ref[...] += jnp.dot(og, hb, precision=HI,
                               preferred_element_type=F32)

    return pl.pallas_call(
        body,
        grid=(nb,),
        in_specs=[
            pl.BlockSpec((2, BN, h), lambda i: (0, i, 0)),
            pl.BlockSpec((2, BN, h + 16), lambda i: (0, i, 0)),
            pl.BlockSpec((BN, h), lambda i: (i, 0)),
            pl.BlockSpec((BN, 1), lambda i: (i, 0)),
            pl.BlockSpec((1, BN), lambda i: (0, i)),
        ],
        out_specs=pl.BlockSpec((NG, h), lambda i: (0, 0)),
        out_shape=jax.ShapeDtypeStruct((NG, h), F32),
    )(sp, dp, r, nmask, b_row)


def _tc_head(xs0, xs1, xs2, xs3, cnt0, cnt1, l1w, l1b, l2wp, l2bp):
    def body(a0, a1, a2, a3, c0, c1, w1, b1, w2, b2, o_ref):
        c0v = jnp.maximum(c0[...], 1.0)
        c1v = jnp.maximum(c1[...], 1.0)
        hcat = jnp.concatenate(
            [a0[...] / c0v, a1[...] / c0v, a2[...] / c1v, a3[...] / c1v],
            axis=1)
        t = jnp.maximum(jnp.dot(hcat, w1[...], precision=HI,
                                preferred_element_type=F32) + b1[...], 0.0)
        lg = jnp.dot(t, w2[...], precision=HI,
                     preferred_element_type=F32) + b2[...]
        colv = lax.broadcasted_iota(jnp.int32, (NG, 128), 1) < 16
        lgm = jnp.where(colv, lg, -1e30)
        m = jnp.max(lgm, axis=1, keepdims=True)
        p = jnp.where(colv, jnp.exp(lgm - m), 0.0)
        lse = jnp.log(jnp.sum(p, axis=1, keepdims=True))
        o_ref[...] = lgm - m - lse

    return pl.pallas_call(
        body,
        out_shape=jax.ShapeDtypeStruct((NG, 128), F32),
    )(xs0, xs1, xs2, xs3, cnt0, cnt1, l1w, l1b, l2wp, l2bp)


def kernel(x, edge_index, batch, c1_wr, c1_wro, c1_b, c2_wr, c2_wro, c2_b,
           c3_wr, c3_wro, c3_b, c4_wr, c4_wro, c4_b, p0_w, p1_w,
           l1_w, l1_b, l2_w, l2_b):
    n, f = x.shape
    h = c1_wr.shape[1]
    npad = ((n + BNJ - 1) // BNJ) * BNJ
    pad = npad - n

    xp = jnp.pad(x, ((0, pad), (0, 0)))
    bf = jnp.pad(batch.astype(F32), (0, pad), constant_values=1e9)
    b_row = bf.reshape(1, npad)
    b_col = bf.reshape(npad, 1)
    eb = 125
    src2d = edge_index[0].reshape(-1, eb)
    dst2d = edge_index[1].reshape(-1, eb)
    zeros80 = jnp.zeros((npad, h + 16), F32)
    zeros64 = zeros80[:, :h]

    y1, r1 = _tc_pre(xp, c1_wr, c1_wro, c1_b.reshape(1, h))
    s1p = _edge_rowagg(y1, src2d, dst2d, zeros80)
    y2, r2, xs0, cnt0 = _tc_combine1(s1p, r1, b_row, c2_wr, c2_wro,
                                     c2_b.reshape(1, h))
    s2p = _edge_rowagg(y2, src2d, dst2d, zeros64)
    h2, z, xs1 = _tc_combine2(s2p, s1p, r2, b_row, p0_w.reshape(1, h))
    rank = _tc_rank(z, z.reshape(1, npad), b_col, b_row)
    y3, r3, nmask = _tc_topk(h2, z, rank, cnt0.reshape(1, NG), b_col,
                             c3_wr, c3_wro, c3_b.reshape(1, h))
    s3p = _edge_rowagg(y3, src2d, dst2d, zeros80)
    y4, r4, xs2, cnt1 = _tc_combine3(s3p, r3, nmask, b_row,
                                     c4_wr, c4_wro, c4_b.reshape(1, h))
    s4p = _edge_rowagg(y4, src2d, dst2d, zeros64)
    xs3 = _tc_combine4(s4p, s3p, r4, nmask, b_row)

    c = l2_w.shape[1]
    l2wp = jnp.pad(l2_w, ((0, 0), (0, 128 - c)))
    l2bp = jnp.pad(l2_b.reshape(1, c), ((0, 0), (0, 128 - c)))
    out = _tc_head(xs0, xs1, xs2, xs3, cnt0, cnt1, l1_w,
                   l1_b.reshape(1, h), l2wp, l2bp)
    return out[:, :c]


# speedup vs baseline: 15.2438x; 1.2462x over previous
import functools

import jax
import jax.numpy as jnp
from jax import lax
from jax.experimental import pallas as pl
from jax.experimental.pallas import tpu as pltpu
from jax.experimental.pallas import tpu_sc as plsc

F32 = jnp.float32
BN = 512
BNJ = 2048
NG = 128
HI = jax.lax.Precision.HIGHEST


def _edge_rowagg(table, src2d, dst2d, zeros):
    npad, w = table.shape
    b = src2d.shape[1]
    e = src2d.shape[0] * b
    nc, ns = 2, 16
    nblk = e // (nc * ns * b)
    rps = npad // ns
    assert e % (nc * ns * b) == 0 and npad % ns == 0 and rps % 8 == 0

    mesh = plsc.VectorSubcoreMesh(core_axis_name="c", subcore_axis_name="s")

    @functools.partial(
        pl.kernel,
        out_type=jax.ShapeDtypeStruct((nc * npad, w), F32),
        mesh=mesh,
        scratch_types=[
            pltpu.VMEM_SHARED((npad, w), F32),
            pltpu.VMEM((nblk, b), jnp.int32),
            pltpu.VMEM((nblk, b), jnp.int32),
            pltpu.VMEM((2, b, w), F32),
            pltpu.SemaphoreType.DMA,
        ],
        compiler_params=pltpu.CompilerParams(use_tc_tiling_on_sc=False),
    )
    def agg(tab_hbm, src_hbm, dst_hbm, z_hbm, out_hbm, acc, src_v,
            dst_v, rows, sem_g):
        c = lax.axis_index("c")
        s = lax.axis_index("s")
        r0 = pl.multiple_of(s * rps, 8)
        pltpu.sync_copy(z_hbm.at[pl.ds(r0, rps)], acc.at[pl.ds(r0, rps)])
        blk0 = (c * ns + s) * nblk
        pltpu.sync_copy(src_hbm.at[pl.ds(blk0, nblk)], src_v)
        pltpu.sync_copy(dst_hbm.at[pl.ds(blk0, nblk)], dst_v)
        plsc.subcore_barrier()

        pltpu.async_copy(tab_hbm.at[src_v.at[0]], rows.at[0], sem_g)

        def body(j, carry):
            jm = lax.rem(j, 2)
            pltpu.make_async_copy(tab_hbm.at[src_v.at[j]], rows.at[jm],
                                  sem_g).wait()

            @pl.when(j + 1 < nblk)
            def _():
                pltpu.async_copy(tab_hbm.at[src_v.at[j + 1]],
                                 rows.at[1 - jm], sem_g)

            pltpu.sync_copy(rows.at[jm], acc.at[dst_v.at[j]], add=True)
            return carry

        lax.fori_loop(0, nblk, body, 0)
        plsc.subcore_barrier()
        o0 = pl.multiple_of(c * npad + s * rps, 8)
        pltpu.sync_copy(acc.at[pl.ds(r0, rps)], out_hbm.at[pl.ds(o0, rps)])

    return agg(table, src2d, dst2d, zeros).reshape(nc, npad, w)


def _tc_pre(xp, w1, w2, bias):
    npad, f = xp.shape
    h = w1.shape[1]
    nb = npad // BN

    def body(x_ref, w1_ref, w2_ref, b_ref, y_ref, r_ref):
        xb = x_ref[...]
        y = jnp.dot(xb, w1_ref[...], precision=HI,
                    preferred_element_type=F32)
        y_ref[...] = jnp.concatenate([y, jnp.ones((BN, 16), F32)], axis=1)
        r_ref[...] = jnp.dot(xb, w2_ref[...], precision=HI,
                             preferred_element_type=F32) + b_ref[...]

    return pl.pallas_call(
        body,
        grid=(nb,),
        in_specs=[
            pl.BlockSpec((BN, f), lambda i: (i, 0)),
            pl.BlockSpec((f, h), lambda i: (0, 0)),
            pl.BlockSpec((f, h), lambda i: (0, 0)),
            pl.BlockSpec((1, h), lambda i: (0, 0)),
        ],
        out_specs=[
            pl.BlockSpec((BN, h + 16), lambda i: (i, 0)),
            pl.BlockSpec((BN, h), lambda i: (i, 0)),
        ],
        out_shape=[
            jax.ShapeDtypeStruct((npad, h + 16), F32),
            jax.ShapeDtypeStruct((npad, h), F32),
        ],
    )(xp, w1, w2, bias)


def _combine_h(sp_ref, dp_ref, r_ref, nm):
    sfull = sp_ref[0] + sp_ref[1]
    if dp_ref is None:
        s = sfull[:, 0:64]
        d = sfull[:, 64:65]
    else:
        s = sfull
        d = dp_ref[0][:, 64:65] + dp_ref[1][:, 64:65]
    if nm is None:
        return jnp.maximum(s / jnp.maximum(d, 1.0) + r_ref[...], 0.0)
    return jnp.maximum(s * nm / jnp.maximum(d * nm, 1.0) + r_ref[...],
                       0.0) * nm


def _onehot(br):
    gcol = lax.broadcasted_iota(jnp.int32, (NG, 1), 0).astype(F32)
    return jnp.where(gcol == br, 1.0, 0.0)


def _tc_combine1(sp, r, b_row, w1n, w2n, bn):
    npad, h = r.shape
    nb = npad // BN

    def body(sp_ref, r_ref, br_ref, w1_ref, w2_ref, bn_ref,
             y_ref, r2_ref, xs_ref, cnt_ref):
        i = pl.program_id(0)
        hb = _combine_h(sp_ref, None, r_ref, None)
        y_ref[...] = jnp.dot(hb, w1_ref[...], precision=HI,
                             preferred_element_type=F32)
        r2_ref[...] = jnp.dot(hb, w2_ref[...], precision=HI,
                              preferred_element_type=F32) + bn_ref[...]
        og = _onehot(br_ref[...])

        @pl.when(i == 0)
        def _():
            xs_ref[...] = jnp.zeros_like(xs_ref)
            cnt_ref[...] = jnp.zeros_like(cnt_ref)

        xs_ref[...] += jnp.dot(og, hb, precision=HI,
                               preferred_element_type=F32)
        cnt_ref[...] += jnp.sum(og, axis=1, keepdims=True)

    return pl.pallas_call(
        body,
        grid=(nb,),
        in_specs=[
            pl.BlockSpec((2, BN, h + 16), lambda i: (0, i, 0)),
            pl.BlockSpec((BN, h), lambda i: (i, 0)),
            pl.BlockSpec((1, BN), lambda i: (0, i)),
            pl.BlockSpec((h, h), lambda i: (0, 0)),
            pl.BlockSpec((h, h), lambda i: (0, 0)),
            pl.BlockSpec((1, h), lambda i: (0, 0)),
        ],
        out_specs=[
            pl.BlockSpec((BN, h), lambda i: (i, 0)),
            pl.BlockSpec((BN, h), lambda i: (i, 0)),
            pl.BlockSpec((NG, h), lambda i: (0, 0)),
            pl.BlockSpec((NG, 1), lambda i: (0, 0)),
        ],
        out_shape=[
            jax.ShapeDtypeStruct((npad, h), F32),
            jax.ShapeDtypeStruct((npad, h), F32),
            jax.ShapeDtypeStruct((NG, h), F32),
            jax.ShapeDtypeStruct((NG, 1), F32),
        ],
    )(sp, r, b_row, w1n, w2n, bn)


def _tc_combine2(sp, dp, r, b_row, pw_row):
    npad, h = r.shape
    nb = npad // BN

    def body(sp_ref, dp_ref, r_ref, br_ref, pw_ref, h_ref, z_ref, xs_ref):
        i = pl.program_id(0)
        hb = _combine_h(sp_ref, dp_ref, r_ref, None)
        h_ref[...] = hb
        pw = pw_ref[...]
        nrm = jnp.sqrt(jnp.sum(pw * pw))
        z_ref[...] = jnp.sum(hb * pw, axis=1, keepdims=True) / nrm
        og = _onehot(br_ref[...])

        @pl.when(i == 0)
        def _():
            xs_ref[...] = jnp.zeros_like(xs_ref)

        xs_ref[...] += jnp.dot(og, hb, precision=HI,
                               preferred_element_type=F32)

    return pl.pallas_call(
        body,
        grid=(nb,),
        in_specs=[
            pl.BlockSpec((2, BN, h), lambda i: (0, i, 0)),
            pl.BlockSpec((2, BN, h + 16), lambda i: (0, i, 0)),
            pl.BlockSpec((BN, h), lambda i: (i, 0)),
            pl.BlockSpec((1, BN), lambda i: (0, i)),
            pl.BlockSpec((1, h), lambda i: (0, 0)),
        ],
        out_specs=[
            pl.BlockSpec((BN, h), lambda i: (i, 0)),
            pl.BlockSpec((BN, 1), lambda i: (i, 0)),
            pl.BlockSpec((NG, h), lambda i: (0, 0)),
        ],
        out_shape=[
            jax.ShapeDtypeStruct((npad, h), F32),
            jax.ShapeDtypeStruct((npad, 1), F32),
            jax.ShapeDtypeStruct((NG, h), F32),
        ],
    )(sp, dp, r, b_row, pw_row)


def _tc_rank(z_col, z_row, b_col, b_row):
    npad = z_col.shape[0]
    nbi, nbj = npad // BN, npad // BNJ

    def body(zi_ref, bi_ref, zj_ref, bj_ref, rk_ref):
        i = pl.program_id(0)
        j = pl.program_id(1)

        @pl.when(j == 0)
        def _():
            rk_ref[...] = jnp.zeros_like(rk_ref)

        bi = bi_ref[...]
        bj = bj_ref[...]
        overlap = jnp.logical_and(bj[0, 0] <= bi[BN - 1, 0],
                                  bj[0, BNJ - 1] >= bi[0, 0])

        @pl.when(overlap)
        def _():
            zi = zi_ref[...]
            zj = zj_ref[...]
            beq = bi == bj
            gi = (i * BN + lax.broadcasted_iota(jnp.int32, (BN, 1), 0)
                  ).astype(F32)
            gj = (j * BNJ + lax.broadcasted_iota(jnp.int32, (1, BNJ), 1)
                  ).astype(F32)
            ahead = (zj > zi) | ((zj == zi) & (gj < gi))
            cmat = jnp.where(beq & ahead, 1.0, 0.0)
            rk_ref[...] += jnp.sum(cmat, axis=1, keepdims=True)

    return pl.pallas_call(
        body,
        grid=(nbi, nbj),
        in_specs=[
            pl.BlockSpec((BN, 1), lambda i, j: (i, 0)),
            pl.BlockSpec((BN, 1), lambda i, j: (i, 0)),
            pl.BlockSpec((1, BNJ), lambda i, j: (0, j)),
            pl.BlockSpec((1, BNJ), lambda i, j: (0, j)),
        ],
        out_specs=pl.BlockSpec((BN, 1), lambda i, j: (i, 0)),
        out_shape=jax.ShapeDtypeStruct((npad, 1), F32),
    )(z_col, b_col, z_row, b_row)


def _tc_topk(h2, z, rank, cnt_row, b_col, w1n, w2n, bn):
    npad, h = h2.shape
    nb = npad // BN

    def body(h_ref, z_ref, rk_ref, cnt_ref, bc_ref, w1_ref, w2_ref, bn_ref,
             y_ref, r2_ref, nm_ref):
        grow = lax.broadcasted_iota(jnp.int32, (1, NG), 1).astype(F32)
        ogt = jnp.where(bc_ref[...] == grow, 1.0, 0.0)
        k_row = jnp.ceil(0.8 * cnt_ref[...])
        k_node = jnp.sum(ogt * k_row, axis=1, keepdims=True)
        nm = jnp.where(rk_ref[...] < k_node, 1.0, 0.0)
        x3 = h_ref[...] * (jnp.tanh(z_ref[...]) * nm)
        y = jnp.dot(x3, w1_ref[...], precision=HI,
                    preferred_element_type=F32)
        y_ref[...] = jnp.concatenate(
            [y, jnp.broadcast_to(nm, (BN, 16))], axis=1)
        r2_ref[...] = jnp.dot(x3, w2_ref[...], precision=HI,
                              preferred_element_type=F32) + bn_ref[...]
        nm_ref[...] = nm

    return pl.pallas_call(
        body,
        grid=(nb,),
        in_specs=[
            pl.BlockSpec((BN, h), lambda i: (i, 0)),
            pl.BlockSpec((BN, 1), lambda i: (i, 0)),
            pl.BlockSpec((BN, 1), lambda i: (i, 0)),
            pl.BlockSpec((1, NG), lambda i: (0, 0)),
            pl.BlockSpec((BN, 1), lambda i: (i, 0)),
            pl.BlockSpec((h, h), lambda i: (0, 0)),
            pl.BlockSpec((h, h), lambda i: (0, 0)),
            pl.BlockSpec((1, h), lambda i: (0, 0)),
        ],
        out_specs=[
            pl.BlockSpec((BN, h + 16), lambda i: (i, 0)),
            pl.BlockSpec((BN, h), lambda i: (i, 0)),
            pl.BlockSpec((BN, 1), lambda i: (i, 0)),
        ],
        out_shape=[
            jax.ShapeDtypeStruct((npad, h + 16), F32),
            jax.ShapeDtypeStruct((npad, h), F32),
            jax.ShapeDtypeStruct((npad, 1), F32),
        ],
    )(h2, z, rank, cnt_row, b_col, w1n, w2n, bn)


def _tc_combine3(sp, r, nmask, b_row, w1n, w2n, bn):
    npad, h = r.shape
    nb = npad // BN

    def body(sp_ref, r_ref, nm_ref, br_ref, w1_ref, w2_ref, bn_ref,
             y_ref, r2_ref, xs_ref, cnt_ref):
        i = pl.program_id(0)
        nm = nm_ref[...]
        hb = _combine_h(sp_ref, None, r_ref, nm)
        y_ref[...] = jnp.dot(hb, w1_ref[...], precision=HI,
                             preferred_element_type=F32)
        r2_ref[...] = jnp.dot(hb, w2_ref[...], precision=HI,
                              preferred_element_type=F32) + bn_ref[...]
        og = _onehot(br_ref[...])

        @pl.when(i == 0)
        def _():
            xs_ref[...] = jnp.zeros_like(xs_ref)
            cnt_ref[...] = jnp.zeros_like(cnt_ref)

        xs_ref[...] += jnp.dot(og, hb, precision=HI,
                               preferred_element_type=F32)
        cnt_ref[...] += jnp.dot(og, nm, precision=HI,
                                preferred_element_type=F32)

    return pl.pallas_call(
        body,
        grid=(nb,),
        in_specs=[
            pl.BlockSpec((2, BN, h + 16), lambda i: (0, i, 0)),
            pl.BlockSpec((BN, h), lambda i: (i, 0)),
            pl.BlockSpec((BN, 1), lambda i: (i, 0)),
            pl.BlockSpec((1, BN), lambda i: (0, i)),
            pl.BlockSpec((h, h), lambda i: (0, 0)),
            pl.BlockSpec((h, h), lambda i: (0, 0)),
            pl.BlockSpec((1, h), lambda i: (0, 0)),
        ],
        out_specs=[
            pl.BlockSpec((BN, h), lambda i: (i, 0)),
            pl.BlockSpec((BN, h), lambda i: (i, 0)),
            pl.BlockSpec((NG, h), lambda i: (0, 0)),
            pl.BlockSpec((NG, 1), lambda i: (0, 0)),
        ],
        out_shape=[
            jax.ShapeDtypeStruct((npad, h), F32),
            jax.ShapeDtypeStruct((npad, h), F32),
            jax.ShapeDtypeStruct((NG, h), F32),
            jax.ShapeDtypeStruct((NG, 1), F32),
        ],
    )(sp, r, nmask, b_row, w1n, w2n, bn)


def _tc_combine4(sp, dp, r, nmask, b_row):
    npad, h = r.shape
    nb = npad // BN

    def body(sp_ref, dp_ref, r_ref, nm_ref, br_ref, xs_ref):
        i = pl.program_id(0)
        hb = _combine_h(sp_ref, dp_ref, r_ref, nm_ref[...])
        og = _onehot(br_ref[...])

        @pl.when(i == 0)
        def _():
            xs_ref[...] = jnp.zeros_like(xs_ref)

        xs_ref[...] += jnp.dot(og, hb, precision=HI,
                               preferred_element_type=F32)

    return pl.pallas_call(
        body,
        grid=(nb,),
        in_specs=[
            pl.BlockSpec((2, BN, h), lambda i: (0, i, 0)),
            pl.BlockSpec((2, BN, h + 16), lambda i: (0, i, 0)),
            pl.BlockSpec((BN, h), lambda i: (i, 0)),
            pl.BlockSpec((BN, 1), lambda i: (i, 0)),
            pl.BlockSpec((1, BN), lambda i: (0, i)),
        ],
        out_specs=pl.BlockSpec((NG, h), lambda i: (0, 0)),
        out_shape=jax.ShapeDtypeStruct((NG, h), F32),
    )(sp, dp, r, nmask, b_row)


def _tc_head(xs0, xs1, xs2, xs3, cnt0, cnt1, l1w, l1b, l2wp, l2bp):
    def body(a0, a1, a2, a3, c0, c1, w1, b1, w2, b2, o_ref):
        c0v = jnp.maximum(c0[...], 1.0)
        c1v = jnp.maximum(c1[...], 1.0)
        hcat = jnp.concatenate(
            [a0[...] / c0v, a1[...] / c0v, a2[...] / c1v, a3[...] / c1v],
            axis=1)
        t = jnp.maximum(jnp.dot(hcat, w1[...], precision=HI,
                                preferred_element_type=F32) + b1[...], 0.0)
        lg = jnp.dot(t, w2[...], precision=HI,
                     preferred_element_type=F32) + b2[...]
        colv = lax.broadcasted_iota(jnp.int32, (NG, 128), 1) < 16
        lgm = jnp.where(colv, lg, -1e30)
        m = jnp.max(lgm, axis=1, keepdims=True)
        p = jnp.where(colv, jnp.exp(lgm - m), 0.0)
        lse = jnp.log(jnp.sum(p, axis=1, keepdims=True))
        o_ref[...] = lgm - m - lse

    return pl.pallas_call(
        body,
        out_shape=jax.ShapeDtypeStruct((NG, 128), F32),
    )(xs0, xs1, xs2, xs3, cnt0, cnt1, l1w, l1b, l2wp, l2bp)


def kernel(x, edge_index, batch, c1_wr, c1_wro, c1_b, c2_wr, c2_wro, c2_b,
           c3_wr, c3_wro, c3_b, c4_wr, c4_wro, c4_b, p0_w, p1_w,
           l1_w, l1_b, l2_w, l2_b):
    n, f = x.shape
    h = c1_wr.shape[1]
    npad = ((n + BNJ - 1) // BNJ) * BNJ
    pad = npad - n

    xp = jnp.pad(x, ((0, pad), (0, 0)))
    bf = jnp.pad(batch.astype(F32), (0, pad), constant_values=1e9)
    b_row = bf.reshape(1, npad)
    b_col = bf.reshape(npad, 1)
    eb = 125
    src2d = edge_index[0].reshape(-1, eb)
    dst2d = edge_index[1].reshape(-1, eb)
    zeros80 = jnp.zeros((npad, h + 16), F32)
    zeros64 = zeros80[:, :h]

    y1, r1 = _tc_pre(xp, c1_wr, c1_wro, c1_b.reshape(1, h))
    s1p = _edge_rowagg(y1, src2d, dst2d, zeros80)
    y2, r2, xs0, cnt0 = _tc_combine1(s1p, r1, b_row, c2_wr, c2_wro,
                                     c2_b.reshape(1, h))
    s2p = _edge_rowagg(y2, src2d, dst2d, zeros64)
    h2, z, xs1 = _tc_combine2(s2p, s1p, r2, b_row, p0_w.reshape(1, h))
    rank = _tc_rank(z, z.reshape(1, npad), b_col, b_row)
    y3, r3, nmask = _tc_topk(h2, z, rank, cnt0.reshape(1, NG), b_col,
                             c3_wr, c3_wro, c3_b.reshape(1, h))
    s3p = _edge_rowagg(y3, src2d, dst2d, zeros80)
    y4, r4, xs2, cnt1 = _tc_combine3(s3p, r3, nmask, b_row,
                                     c4_wr, c4_wro, c4_b.reshape(1, h))
    s4p = _edge_rowagg(y4, src2d, dst2d, zeros64)
    xs3 = _tc_combine4(s4p, s3p, r4, nmask, b_row)

    c = l2_w.shape[1]
    l2wp = jnp.pad(l2_w, ((0, 0), (0, 128 - c)))
    l2bp = jnp.pad(l2_b.reshape(1, c), ((0, 0), (0, 128 - c)))
    out = _tc_head(xs0, xs1, xs2, xs3, cnt0, cnt1, l1_w,
                   l1_b.reshape(1, h), l2wp, l2bp)
    return out[:, :c]


# speedup vs baseline: 16.1817x; 1.0615x over previous
import functools

import jax
import jax.numpy as jnp
from jax import lax
from jax.experimental import pallas as pl
from jax.experimental.pallas import tpu as pltpu
from jax.experimental.pallas import tpu_sc as plsc

F32 = jnp.float32
BN = 1024
BNJ = 2048
NG = 128
HI = jax.lax.Precision.HIGHEST


def _edge_rowagg(table, src2d, dst2d, zeros):
    npad, w = table.shape
    b = src2d.shape[1]
    e = src2d.shape[0] * b
    nc, ns = 2, 16
    nblk = e // (nc * ns * b)
    rps = npad // ns
    assert e % (nc * ns * b) == 0 and npad % ns == 0 and rps % 8 == 0

    mesh = plsc.VectorSubcoreMesh(core_axis_name="c", subcore_axis_name="s")

    @functools.partial(
        pl.kernel,
        out_type=jax.ShapeDtypeStruct((nc * npad, w), F32),
        mesh=mesh,
        scratch_types=[
            pltpu.VMEM_SHARED((npad, w), F32),
            pltpu.VMEM((nblk, b), jnp.int32),
            pltpu.VMEM((nblk, b), jnp.int32),
            pltpu.VMEM((2, b, w), F32),
            pltpu.SemaphoreType.DMA,
        ],
        compiler_params=pltpu.CompilerParams(use_tc_tiling_on_sc=False),
    )
    def agg(tab_hbm, src_hbm, dst_hbm, z_hbm, out_hbm, acc, src_v,
            dst_v, rows, sem_g):
        c = lax.axis_index("c")
        s = lax.axis_index("s")
        r0 = pl.multiple_of(s * rps, 8)
        pltpu.sync_copy(z_hbm.at[pl.ds(r0, rps)], acc.at[pl.ds(r0, rps)])
        blk0 = (c * ns + s) * nblk
        pltpu.sync_copy(src_hbm.at[pl.ds(blk0, nblk)], src_v)
        pltpu.sync_copy(dst_hbm.at[pl.ds(blk0, nblk)], dst_v)
        plsc.subcore_barrier()

        pltpu.async_copy(tab_hbm.at[src_v.at[0]], rows.at[0], sem_g)

        def body(j, carry):
            jm = lax.rem(j, 2)
            pltpu.make_async_copy(tab_hbm.at[src_v.at[j]], rows.at[jm],
                                  sem_g).wait()

            @pl.when(j + 1 < nblk)
            def _():
                pltpu.async_copy(tab_hbm.at[src_v.at[j + 1]],
                                 rows.at[1 - jm], sem_g)

            pltpu.sync_copy(rows.at[jm], acc.at[dst_v.at[j]], add=True)
            return carry

        lax.fori_loop(0, nblk, body, 0)
        plsc.subcore_barrier()
        o0 = pl.multiple_of(c * npad + s * rps, 8)
        pltpu.sync_copy(acc.at[pl.ds(r0, rps)], out_hbm.at[pl.ds(o0, rps)])

    return agg(table, src2d, dst2d, zeros).reshape(nc, npad, w)


def _tc_pre(xp, w1, w2, bias):
    npad, f = xp.shape
    h = w1.shape[1]
    nb = npad // BN

    def body(x_ref, w1_ref, w2_ref, b_ref, y_ref, r_ref):
        xb = x_ref[...]
        y = jnp.dot(xb, w1_ref[...], precision=HI,
                    preferred_element_type=F32)
        y_ref[...] = jnp.concatenate([y, jnp.ones((BN, 16), F32)], axis=1)
        r_ref[...] = jnp.dot(xb, w2_ref[...], precision=HI,
                             preferred_element_type=F32) + b_ref[...]

    return pl.pallas_call(
        body,
        grid=(nb,),
        in_specs=[
            pl.BlockSpec((BN, f), lambda i: (i, 0)),
            pl.BlockSpec((f, h), lambda i: (0, 0)),
            pl.BlockSpec((f, h), lambda i: (0, 0)),
            pl.BlockSpec((1, h), lambda i: (0, 0)),
        ],
        out_specs=[
            pl.BlockSpec((BN, h + 16), lambda i: (i, 0)),
            pl.BlockSpec((BN, h), lambda i: (i, 0)),
        ],
        out_shape=[
            jax.ShapeDtypeStruct((npad, h + 16), F32),
            jax.ShapeDtypeStruct((npad, h), F32),
        ],
    )(xp, w1, w2, bias)


def _combine_h(sp_ref, dp_ref, r_ref, nm):
    sfull = sp_ref[0] + sp_ref[1]
    if dp_ref is None:
        s = sfull[:, 0:64]
        d = sfull[:, 64:65]
    else:
        s = sfull
        d = dp_ref[0][:, 64:65] + dp_ref[1][:, 64:65]
    if nm is None:
        return jnp.maximum(s / jnp.maximum(d, 1.0) + r_ref[...], 0.0)
    return jnp.maximum(s * nm / jnp.maximum(d * nm, 1.0) + r_ref[...],
                       0.0) * nm


def _onehot(br):
    gcol = lax.broadcasted_iota(jnp.int32, (NG, 1), 0).astype(F32)
    return jnp.where(gcol == br, 1.0, 0.0)


def _tc_combine1(sp, r, b_row, w1n, w2n, bn):
    npad, h = r.shape
    nb = npad // BN

    def body(sp_ref, r_ref, br_ref, w1_ref, w2_ref, bn_ref,
             y_ref, r2_ref, xs_ref, cnt_ref):
        i = pl.program_id(0)
        hb = _combine_h(sp_ref, None, r_ref, None)
        y_ref[...] = jnp.dot(hb, w1_ref[...], precision=HI,
                             preferred_element_type=F32)
        r2_ref[...] = jnp.dot(hb, w2_ref[...], precision=HI,
                              preferred_element_type=F32) + bn_ref[...]
        og = _onehot(br_ref[...])

        @pl.when(i == 0)
        def _():
            xs_ref[...] = jnp.zeros_like(xs_ref)
            cnt_ref[...] = jnp.zeros_like(cnt_ref)

        xs_ref[...] += jnp.dot(og, hb, precision=HI,
                               preferred_element_type=F32)
        cnt_ref[...] += jnp.sum(og, axis=1, keepdims=True)

    return pl.pallas_call(
        body,
        grid=(nb,),
        in_specs=[
            pl.BlockSpec((2, BN, h + 16), lambda i: (0, i, 0)),
            pl.BlockSpec((BN, h), lambda i: (i, 0)),
            pl.BlockSpec((1, BN), lambda i: (0, i)),
            pl.BlockSpec((h, h), lambda i: (0, 0)),
            pl.BlockSpec((h, h), lambda i: (0, 0)),
            pl.BlockSpec((1, h), lambda i: (0, 0)),
        ],
        out_specs=[
            pl.BlockSpec((BN, h), lambda i: (i, 0)),
            pl.BlockSpec((BN, h), lambda i: (i, 0)),
            pl.BlockSpec((NG, h), lambda i: (0, 0)),
            pl.BlockSpec((NG, 1), lambda i: (0, 0)),
        ],
        out_shape=[
            jax.ShapeDtypeStruct((npad, h), F32),
            jax.ShapeDtypeStruct((npad, h), F32),
            jax.ShapeDtypeStruct((NG, h), F32),
            jax.ShapeDtypeStruct((NG, 1), F32),
        ],
    )(sp, r, b_row, w1n, w2n, bn)


def _tc_combine2(sp, dp, r, b_row, pw_row):
    npad, h = r.shape
    nb = npad // BN

    def body(sp_ref, dp_ref, r_ref, br_ref, pw_ref, h_ref, z_ref, xs_ref):
        i = pl.program_id(0)
        hb = _combine_h(sp_ref, dp_ref, r_ref, None)
        h_ref[...] = hb
        pw = pw_ref[...]
        nrm = jnp.sqrt(jnp.sum(pw * pw))
        z_ref[...] = jnp.sum(hb * pw, axis=1, keepdims=True) / nrm
        og = _onehot(br_ref[...])

        @pl.when(i == 0)
        def _():
            xs_ref[...] = jnp.zeros_like(xs_ref)

        xs_ref[...] += jnp.dot(og, hb, precision=HI,
                               preferred_element_type=F32)

    return pl.pallas_call(
        body,
        grid=(nb,),
        in_specs=[
            pl.BlockSpec((2, BN, h), lambda i: (0, i, 0)),
            pl.BlockSpec((2, BN, h + 16), lambda i: (0, i, 0)),
            pl.BlockSpec((BN, h), lambda i: (i, 0)),
            pl.BlockSpec((1, BN), lambda i: (0, i)),
            pl.BlockSpec((1, h), lambda i: (0, 0)),
        ],
        out_specs=[
            pl.BlockSpec((BN, h), lambda i: (i, 0)),
            pl.BlockSpec((BN, 1), lambda i: (i, 0)),
            pl.BlockSpec((NG, h), lambda i: (0, 0)),
        ],
        out_shape=[
            jax.ShapeDtypeStruct((npad, h), F32),
            jax.ShapeDtypeStruct((npad, 1), F32),
            jax.ShapeDtypeStruct((NG, h), F32),
        ],
    )(sp, dp, r, b_row, pw_row)


def _tc_rank(z_col, z_row, b_col, b_row):
    npad = z_col.shape[0]
    nbi, nbj = npad // BN, npad // BNJ

    def body(zi_ref, bi_ref, zj_ref, bj_ref, rk_ref):
        i = pl.program_id(0)
        j = pl.program_id(1)

        @pl.when(j == 0)
        def _():
            rk_ref[...] = jnp.zeros_like(rk_ref)

        bi = bi_ref[...]
        bj = bj_ref[...]
        overlap = jnp.logical_and(bj[0, 0] <= bi[BN - 1, 0],
                                  bj[0, BNJ - 1] >= bi[0, 0])

        @pl.when(overlap)
        def _():
            zi = zi_ref[...]
            zj = zj_ref[...]
            beq = bi == bj
            gi = (i * BN + lax.broadcasted_iota(jnp.int32, (BN, 1), 0)
                  ).astype(F32)
            gj = (j * BNJ + lax.broadcasted_iota(jnp.int32, (1, BNJ), 1)
                  ).astype(F32)
            ahead = (zj > zi) | ((zj == zi) & (gj < gi))
            cmat = jnp.where(beq & ahead, 1.0, 0.0)
            rk_ref[...] += jnp.sum(cmat, axis=1, keepdims=True)

    return pl.pallas_call(
        body,
        grid=(nbi, nbj),
        in_specs=[
            pl.BlockSpec((BN, 1), lambda i, j: (i, 0)),
            pl.BlockSpec((BN, 1), lambda i, j: (i, 0)),
            pl.BlockSpec((1, BNJ), lambda i, j: (0, j)),
            pl.BlockSpec((1, BNJ), lambda i, j: (0, j)),
        ],
        out_specs=pl.BlockSpec((BN, 1), lambda i, j: (i, 0)),
        out_shape=jax.ShapeDtypeStruct((npad, 1), F32),
    )(z_col, b_col, z_row, b_row)


def _tc_topk(h2, z, rank, cnt_row, b_col, w1n, w2n, bn):
    npad, h = h2.shape
    nb = npad // BN

    def body(h_ref, z_ref, rk_ref, cnt_ref, bc_ref, w1_ref, w2_ref, bn_ref,
             y_ref, r2_ref, nm_ref):
        grow = lax.broadcasted_iota(jnp.int32, (1, NG), 1).astype(F32)
        ogt = jnp.where(bc_ref[...] == grow, 1.0, 0.0)
        k_row = jnp.ceil(0.8 * cnt_ref[...])
        k_node = jnp.sum(ogt * k_row, axis=1, keepdims=True)
        nm = jnp.where(rk_ref[...] < k_node, 1.0, 0.0)
        x3 = h_ref[...] * (jnp.tanh(z_ref[...]) * nm)
        y = jnp.dot(x3, w1_ref[...], precision=HI,
                    preferred_element_type=F32)
        y_ref[...] = jnp.concatenate(
            [y, jnp.broadcast_to(nm, (BN, 16))], axis=1)
        r2_ref[...] = jnp.dot(x3, w2_ref[...], precision=HI,
                              preferred_element_type=F32) + bn_ref[...]
        nm_ref[...] = nm

    return pl.pallas_call(
        body,
        grid=(nb,),
        in_specs=[
            pl.BlockSpec((BN, h), lambda i: (i, 0)),
            pl.BlockSpec((BN, 1), lambda i: (i, 0)),
            pl.BlockSpec((BN, 1), lambda i: (i, 0)),
            pl.BlockSpec((1, NG), lambda i: (0, 0)),
            pl.BlockSpec((BN, 1), lambda i: (i, 0)),
            pl.BlockSpec((h, h), lambda i: (0, 0)),
            pl.BlockSpec((h, h), lambda i: (0, 0)),
            pl.BlockSpec((1, h), lambda i: (0, 0)),
        ],
        out_specs=[
            pl.BlockSpec((BN, h + 16), lambda i: (i, 0)),
            pl.BlockSpec((BN, h), lambda i: (i, 0)),
            pl.BlockSpec((BN, 1), lambda i: (i, 0)),
        ],
        out_shape=[
            jax.ShapeDtypeStruct((npad, h + 16), F32),
            jax.ShapeDtypeStruct((npad, h), F32),
            jax.ShapeDtypeStruct((npad, 1), F32),
        ],
    )(h2, z, rank, cnt_row, b_col, w1n, w2n, bn)


def _tc_combine3(sp, r, nmask, b_row, w1n, w2n, bn):
    npad, h = r.shape
    nb = npad // BN

    def body(sp_ref, r_ref, nm_ref, br_ref, w1_ref, w2_ref, bn_ref,
             y_ref, r2_ref, xs_ref, cnt_ref):
        i = pl.program_id(0)
        nm = nm_ref[...]
        hb = _combine_h(sp_ref, None, r_ref, nm)
        y_ref[...] = jnp.dot(hb, w1_ref[...], precision=HI,
                             preferred_element_type=F32)
        r2_ref[...] = jnp.dot(hb, w2_ref[...], precision=HI,
                              preferred_element_type=F32) + bn_ref[...]
        og = _onehot(br_ref[...])

        @pl.when(i == 0)
        def _():
            xs_ref[...] = jnp.zeros_like(xs_ref)
            cnt_ref[...] = jnp.zeros_like(cnt_ref)

        xs_ref[...] += jnp.dot(og, hb, precision=HI,
                               preferred_element_type=F32)
        cnt_ref[...] += jnp.dot(og, nm, precision=HI,
                                preferred_element_type=F32)

    return pl.pallas_call(
        body,
        grid=(nb,),
        in_specs=[
            pl.BlockSpec((2, BN, h + 16), lambda i: (0, i, 0)),
            pl.BlockSpec((BN, h), lambda i: (i, 0)),
            pl.BlockSpec((BN, 1), lambda i: (i, 0)),
            pl.BlockSpec((1, BN), lambda i: (0, i)),
            pl.BlockSpec((h, h), lambda i: (0, 0)),
            pl.BlockSpec((h, h), lambda i: (0, 0)),
            pl.BlockSpec((1, h), lambda i: (0, 0)),
        ],
        out_specs=[
            pl.BlockSpec((BN, h), lambda i: (i, 0)),
            pl.BlockSpec((BN, h), lambda i: (i, 0)),
            pl.BlockSpec((NG, h), lambda i: (0, 0)),
            pl.BlockSpec((NG, 1), lambda i: (0, 0)),
        ],
        out_shape=[
            jax.ShapeDtypeStruct((npad, h), F32),
            jax.ShapeDtypeStruct((npad, h), F32),
            jax.ShapeDtypeStruct((NG, h), F32),
            jax.ShapeDtypeStruct((NG, 1), F32),
        ],
    )(sp, r, nmask, b_row, w1n, w2n, bn)


def _tc_combine4(sp, dp, r, nmask, b_row):
    npad, h = r.shape
    nb = npad // BN

    def body(sp_ref, dp_ref, r_ref, nm_ref, br_ref, xs_ref):
        i = pl.program_id(0)
        hb = _combine_h(sp_ref, dp_ref, r_ref, nm_ref[...])
        og = _onehot(br_ref[...])

        @pl.when(i == 0)
        def _():
            xs_ref[...] = jnp.zeros_like(xs_ref)

        xs_ref[...] += jnp.dot(og, hb, precision=HI,
                               preferred_element_type=F32)

    return pl.pallas_call(
        body,
        grid=(nb,),
        in_specs=[
            pl.BlockSpec((2, BN, h), lambda i: (0, i, 0)),
            pl.BlockSpec((2, BN, h + 16), lambda i: (0, i, 0)),
            pl.BlockSpec((BN, h), lambda i: (i, 0)),
            pl.BlockSpec((BN, 1), lambda i: (i, 0)),
            pl.BlockSpec((1, BN), lambda i: (0, i)),
        ],
        out_specs=pl.BlockSpec((NG, h), lambda i: (0, 0)),
        out_shape=jax.ShapeDtypeStruct((NG, h), F32),
    )(sp, dp, r, nmask, b_row)


def _tc_head(xs0, xs1, xs2, xs3, cnt0, cnt1, l1w, l1b, l2wp, l2bp):
    def body(a0, a1, a2, a3, c0, c1, w1, b1, w2, b2, o_ref):
        c0v = jnp.maximum(c0[...], 1.0)
        c1v = jnp.maximum(c1[...], 1.0)
        hcat = jnp.concatenate(
            [a0[...] / c0v, a1[...] / c0v, a2[...] / c1v, a3[...] / c1v],
            axis=1)
        t = jnp.maximum(jnp.dot(hcat, w1[...], precision=HI,
                                preferred_element_type=F32) + b1[...], 0.0)
        lg = jnp.dot(t, w2[...], precision=HI,
                     preferred_element_type=F32) + b2[...]
        colv = lax.broadcasted_iota(jnp.int32, (NG, 128), 1) < 16
        lgm = jnp.where(colv, lg, -1e30)
        m = jnp.max(lgm, axis=1, keepdims=True)
        p = jnp.where(colv, jnp.exp(lgm - m), 0.0)
        lse = jnp.log(jnp.sum(p, axis=1, keepdims=True))
        o_ref[...] = lgm - m - lse

    return pl.pallas_call(
        body,
        out_shape=jax.ShapeDtypeStruct((NG, 128), F32),
    )(xs0, xs1, xs2, xs3, cnt0, cnt1, l1w, l1b, l2wp, l2bp)


def kernel(x, edge_index, batch, c1_wr, c1_wro, c1_b, c2_wr, c2_wro, c2_b,
           c3_wr, c3_wro, c3_b, c4_wr, c4_wro, c4_b, p0_w, p1_w,
           l1_w, l1_b, l2_w, l2_b):
    n, f = x.shape
    h = c1_wr.shape[1]
    npad = ((n + BNJ - 1) // BNJ) * BNJ
    pad = npad - n

    xp = jnp.pad(x, ((0, pad), (0, 0)))
    bf = jnp.pad(batch.astype(F32), (0, pad), constant_values=1e9)
    b_row = bf.reshape(1, npad)
    b_col = bf.reshape(npad, 1)
    eb = 125
    src2d = edge_index[0].reshape(-1, eb)
    dst2d = edge_index[1].reshape(-1, eb)
    zeros80 = jnp.zeros((npad, h + 16), F32)
    zeros64 = zeros80[:, :h]

    y1, r1 = _tc_pre(xp, c1_wr, c1_wro, c1_b.reshape(1, h))
    s1p = _edge_rowagg(y1, src2d, dst2d, zeros80)
    y2, r2, xs0, cnt0 = _tc_combine1(s1p, r1, b_row, c2_wr, c2_wro,
                                     c2_b.reshape(1, h))
    s2p = _edge_rowagg(y2, src2d, dst2d, zeros64)
    h2, z, xs1 = _tc_combine2(s2p, s1p, r2, b_row, p0_w.reshape(1, h))
    rank = _tc_rank(z, z.reshape(1, npad), b_col, b_row)
    y3, r3, nmask = _tc_topk(h2, z, rank, cnt0.reshape(1, NG), b_col,
                             c3_wr, c3_wro, c3_b.reshape(1, h))
    s3p = _edge_rowagg(y3, src2d, dst2d, zeros80)
    y4, r4, xs2, cnt1 = _tc_combine3(s3p, r3, nmask, b_row,
                                     c4_wr, c4_wro, c4_b.reshape(1, h))
    s4p = _edge_rowagg(y4, src2d, dst2d, zeros64)
    xs3 = _tc_combine4(s4p, s3p, r4, nmask, b_row)

    c = l2_w.shape[1]
    l2wp = jnp.pad(l2_w, ((0, 0), (0, 128 - c)))
    l2bp = jnp.pad(l2_b.reshape(1, c), ((0, 0), (0, 128 - c)))
    out = _tc_head(xs0, xs1, xs2, xs3, cnt0, cnt1, l1_w,
                   l1_b.reshape(1, h), l2wp, l2bp)
    return out[:, :c]


# speedup vs baseline: 16.9038x; 1.0446x over previous
import functools

import jax
import jax.numpy as jnp
from jax import lax
from jax.experimental import pallas as pl
from jax.experimental.pallas import tpu as pltpu
from jax.experimental.pallas import tpu_sc as plsc

F32 = jnp.float32
BN = 1024
BNJ = 2048
NG = 128
HI = jax.lax.Precision.DEFAULT


def _edge_rowagg(table, src2d, dst2d, zeros):
    npad, w = table.shape
    b = src2d.shape[1]
    e = src2d.shape[0] * b
    nc, ns = 2, 16
    nblk = e // (nc * ns * b)
    rps = npad // ns
    assert e % (nc * ns * b) == 0 and npad % ns == 0 and rps % 8 == 0

    mesh = plsc.VectorSubcoreMesh(core_axis_name="c", subcore_axis_name="s")

    @functools.partial(
        pl.kernel,
        out_type=jax.ShapeDtypeStruct((nc * npad, w), F32),
        mesh=mesh,
        scratch_types=[
            pltpu.VMEM_SHARED((npad, w), F32),
            pltpu.VMEM((nblk, b), jnp.int32),
            pltpu.VMEM((nblk, b), jnp.int32),
            pltpu.VMEM((2, b, w), F32),
            pltpu.SemaphoreType.DMA,
        ],
        compiler_params=pltpu.CompilerParams(use_tc_tiling_on_sc=False),
    )
    def agg(tab_hbm, src_hbm, dst_hbm, z_hbm, out_hbm, acc, src_v,
            dst_v, rows, sem_g):
        c = lax.axis_index("c")
        s = lax.axis_index("s")
        r0 = pl.multiple_of(s * rps, 8)
        pltpu.sync_copy(z_hbm.at[pl.ds(r0, rps)], acc.at[pl.ds(r0, rps)])
        blk0 = (c * ns + s) * nblk
        pltpu.sync_copy(src_hbm.at[pl.ds(blk0, nblk)], src_v)
        pltpu.sync_copy(dst_hbm.at[pl.ds(blk0, nblk)], dst_v)
        plsc.subcore_barrier()

        pltpu.async_copy(tab_hbm.at[src_v.at[0]], rows.at[0], sem_g)

        def body(j, carry):
            jm = lax.rem(j, 2)
            pltpu.make_async_copy(tab_hbm.at[src_v.at[j]], rows.at[jm],
                                  sem_g).wait()

            @pl.when(j + 1 < nblk)
            def _():
                pltpu.async_copy(tab_hbm.at[src_v.at[j + 1]],
                                 rows.at[1 - jm], sem_g)

            pltpu.sync_copy(rows.at[jm], acc.at[dst_v.at[j]], add=True)
            return carry

        lax.fori_loop(0, nblk, body, 0)
        plsc.subcore_barrier()
        o0 = pl.multiple_of(c * npad + s * rps, 8)
        pltpu.sync_copy(acc.at[pl.ds(r0, rps)], out_hbm.at[pl.ds(o0, rps)])

    return agg(table, src2d, dst2d, zeros).reshape(nc, npad, w)


def _tc_pre(xp, w1, w2, bias):
    npad, f = xp.shape
    h = w1.shape[1]
    nb = npad // BN

    def body(x_ref, w1_ref, w2_ref, b_ref, y_ref, r_ref):
        xb = x_ref[...]
        y = jnp.dot(xb, w1_ref[...], precision=HI,
                    preferred_element_type=F32)
        y_ref[...] = jnp.concatenate([y, jnp.ones((BN, 16), F32)], axis=1)
        r_ref[...] = jnp.dot(xb, w2_ref[...], precision=HI,
                             preferred_element_type=F32) + b_ref[...]

    return pl.pallas_call(
        body,
        grid=(nb,),
        in_specs=[
            pl.BlockSpec((BN, f), lambda i: (i, 0)),
            pl.BlockSpec((f, h), lambda i: (0, 0)),
            pl.BlockSpec((f, h), lambda i: (0, 0)),
            pl.BlockSpec((1, h), lambda i: (0, 0)),
        ],
        out_specs=[
            pl.BlockSpec((BN, h + 16), lambda i: (i, 0)),
            pl.BlockSpec((BN, h), lambda i: (i, 0)),
        ],
        out_shape=[
            jax.ShapeDtypeStruct((npad, h + 16), F32),
            jax.ShapeDtypeStruct((npad, h), F32),
        ],
    )(xp, w1, w2, bias)


def _combine_h(sp_ref, dp_ref, r_ref, nm):
    sfull = sp_ref[0] + sp_ref[1]
    if dp_ref is None:
        s = sfull[:, 0:64]
        d = sfull[:, 64:65]
    else:
        s = sfull
        d = dp_ref[0][:, 64:65] + dp_ref[1][:, 64:65]
    if nm is None:
        return jnp.maximum(s / jnp.maximum(d, 1.0) + r_ref[...], 0.0)
    return jnp.maximum(s * nm / jnp.maximum(d * nm, 1.0) + r_ref[...],
                       0.0) * nm


def _onehot(br):
    gcol = lax.broadcasted_iota(jnp.int32, (NG, 1), 0).astype(F32)
    return jnp.where(gcol == br, 1.0, 0.0)


def _tc_combine1(sp, r, b_row, w1n, w2n, bn):
    npad, h = r.shape
    nb = npad // BN

    def body(sp_ref, r_ref, br_ref, w1_ref, w2_ref, bn_ref,
             y_ref, r2_ref, xs_ref, cnt_ref):
        i = pl.program_id(0)
        hb = _combine_h(sp_ref, None, r_ref, None)
        y_ref[...] = jnp.dot(hb, w1_ref[...], precision=HI,
                             preferred_element_type=F32)
        r2_ref[...] = jnp.dot(hb, w2_ref[...], precision=HI,
                              preferred_element_type=F32) + bn_ref[...]
        og = _onehot(br_ref[...])

        @pl.when(i == 0)
        def _():
            xs_ref[...] = jnp.zeros_like(xs_ref)
            cnt_ref[...] = jnp.zeros_like(cnt_ref)

        xs_ref[...] += jnp.dot(og, hb, precision=HI,
                               preferred_element_type=F32)
        cnt_ref[...] += jnp.sum(og, axis=1, keepdims=True)

    return pl.pallas_call(
        body,
        grid=(nb,),
        in_specs=[
            pl.BlockSpec((2, BN, h + 16), lambda i: (0, i, 0)),
            pl.BlockSpec((BN, h), lambda i: (i, 0)),
            pl.BlockSpec((1, BN), lambda i: (0, i)),
            pl.BlockSpec((h, h), lambda i: (0, 0)),
            pl.BlockSpec((h, h), lambda i: (0, 0)),
            pl.BlockSpec((1, h), lambda i: (0, 0)),
        ],
        out_specs=[
            pl.BlockSpec((BN, h), lambda i: (i, 0)),
            pl.BlockSpec((BN, h), lambda i: (i, 0)),
            pl.BlockSpec((NG, h), lambda i: (0, 0)),
            pl.BlockSpec((NG, 1), lambda i: (0, 0)),
        ],
        out_shape=[
            jax.ShapeDtypeStruct((npad, h), F32),
            jax.ShapeDtypeStruct((npad, h), F32),
            jax.ShapeDtypeStruct((NG, h), F32),
            jax.ShapeDtypeStruct((NG, 1), F32),
        ],
    )(sp, r, b_row, w1n, w2n, bn)


def _tc_combine2(sp, dp, r, b_row, pw_row):
    npad, h = r.shape
    nb = npad // BN

    def body(sp_ref, dp_ref, r_ref, br_ref, pw_ref, h_ref, z_ref, xs_ref):
        i = pl.program_id(0)
        hb = _combine_h(sp_ref, dp_ref, r_ref, None)
        h_ref[...] = hb
        pw = pw_ref[...]
        nrm = jnp.sqrt(jnp.sum(pw * pw))
        z_ref[...] = jnp.sum(hb * pw, axis=1, keepdims=True) / nrm
        og = _onehot(br_ref[...])

        @pl.when(i == 0)
        def _():
            xs_ref[...] = jnp.zeros_like(xs_ref)

        xs_ref[...] += jnp.dot(og, hb, precision=HI,
                               preferred_element_type=F32)

    return pl.pallas_call(
        body,
        grid=(nb,),
        in_specs=[
            pl.BlockSpec((2, BN, h), lambda i: (0, i, 0)),
            pl.BlockSpec((2, BN, h + 16), lambda i: (0, i, 0)),
            pl.BlockSpec((BN, h), lambda i: (i, 0)),
            pl.BlockSpec((1, BN), lambda i: (0, i)),
            pl.BlockSpec((1, h), lambda i: (0, 0)),
        ],
        out_specs=[
            pl.BlockSpec((BN, h), lambda i: (i, 0)),
            pl.BlockSpec((BN, 1), lambda i: (i, 0)),
            pl.BlockSpec((NG, h), lambda i: (0, 0)),
        ],
        out_shape=[
            jax.ShapeDtypeStruct((npad, h), F32),
            jax.ShapeDtypeStruct((npad, 1), F32),
            jax.ShapeDtypeStruct((NG, h), F32),
        ],
    )(sp, dp, r, b_row, pw_row)


def _tc_rank(z_col, z_row, b_col, b_row):
    npad = z_col.shape[0]
    nbi, nbj = npad // BN, npad // BNJ

    def body(zi_ref, bi_ref, zj_ref, bj_ref, rk_ref):
        i = pl.program_id(0)
        j = pl.program_id(1)

        @pl.when(j == 0)
        def _():
            rk_ref[...] = jnp.zeros_like(rk_ref)

        bi = bi_ref[...]
        bj = bj_ref[...]
        overlap = jnp.logical_and(bj[0, 0] <= bi[BN - 1, 0],
                                  bj[0, BNJ - 1] >= bi[0, 0])

        @pl.when(overlap)
        def _():
            zi = zi_ref[...]
            zj = zj_ref[...]
            beq = bi == bj
            gi = (i * BN + lax.broadcasted_iota(jnp.int32, (BN, 1), 0)
                  ).astype(F32)
            gj = (j * BNJ + lax.broadcasted_iota(jnp.int32, (1, BNJ), 1)
                  ).astype(F32)
            ahead = (zj > zi) | ((zj == zi) & (gj < gi))
            cmat = jnp.where(beq & ahead, 1.0, 0.0)
            rk_ref[...] += jnp.sum(cmat, axis=1, keepdims=True)

    return pl.pallas_call(
        body,
        grid=(nbi, nbj),
        in_specs=[
            pl.BlockSpec((BN, 1), lambda i, j: (i, 0)),
            pl.BlockSpec((BN, 1), lambda i, j: (i, 0)),
            pl.BlockSpec((1, BNJ), lambda i, j: (0, j)),
            pl.BlockSpec((1, BNJ), lambda i, j: (0, j)),
        ],
        out_specs=pl.BlockSpec((BN, 1), lambda i, j: (i, 0)),
        out_shape=jax.ShapeDtypeStruct((npad, 1), F32),
    )(z_col, b_col, z_row, b_row)


def _tc_topk(h2, z, rank, cnt_row, b_col, w1n, w2n, bn):
    npad, h = h2.shape
    nb = npad // BN

    def body(h_ref, z_ref, rk_ref, cnt_ref, bc_ref, w1_ref, w2_ref, bn_ref,
             y_ref, r2_ref, nm_ref):
        grow = lax.broadcasted_iota(jnp.int32, (1, NG), 1).astype(F32)
        ogt = jnp.where(bc_ref[...] == grow, 1.0, 0.0)
        k_row = jnp.ceil(0.8 * cnt_ref[...])
        k_node = jnp.sum(ogt * k_row, axis=1, keepdims=True)
        nm = jnp.where(rk_ref[...] < k_node, 1.0, 0.0)
        x3 = h_ref[...] * (jnp.tanh(z_ref[...]) * nm)
        y = jnp.dot(x3, w1_ref[...], precision=HI,
                    preferred_element_type=F32)
        y_ref[...] = jnp.concatenate(
            [y, jnp.broadcast_to(nm, (BN, 16))], axis=1)
        r2_ref[...] = jnp.dot(x3, w2_ref[...], precision=HI,
                              preferred_element_type=F32) + bn_ref[...]
        nm_ref[...] = nm

    return pl.pallas_call(
        body,
        grid=(nb,),
        in_specs=[
            pl.BlockSpec((BN, h), lambda i: (i, 0)),
            pl.BlockSpec((BN, 1), lambda i: (i, 0)),
            pl.BlockSpec((BN, 1), lambda i: (i, 0)),
            pl.BlockSpec((1, NG), lambda i: (0, 0)),
            pl.BlockSpec((BN, 1), lambda i: (i, 0)),
            pl.BlockSpec((h, h), lambda i: (0, 0)),
            pl.BlockSpec((h, h), lambda i: (0, 0)),
            pl.BlockSpec((1, h), lambda i: (0, 0)),
        ],
        out_specs=[
            pl.BlockSpec((BN, h + 16), lambda i: (i, 0)),
            pl.BlockSpec((BN, h), lambda i: (i, 0)),
            pl.BlockSpec((BN, 1), lambda i: (i, 0)),
        ],
        out_shape=[
            jax.ShapeDtypeStruct((npad, h + 16), F32),
            jax.ShapeDtypeStruct((npad, h), F32),
            jax.ShapeDtypeStruct((npad, 1), F32),
        ],
    )(h2, z, rank, cnt_row, b_col, w1n, w2n, bn)


def _tc_combine3(sp, r, nmask, b_row, w1n, w2n, bn):
    npad, h = r.shape
    nb = npad // BN

    def body(sp_ref, r_ref, nm_ref, br_ref, w1_ref, w2_ref, bn_ref,
             y_ref, r2_ref, xs_ref, cnt_ref):
        i = pl.program_id(0)
        nm = nm_ref[...]
        hb = _combine_h(sp_ref, None, r_ref, nm)
        y_ref[...] = jnp.dot(hb, w1_ref[...], precision=HI,
                             preferred_element_type=F32)
        r2_ref[...] = jnp.dot(hb, w2_ref[...], precision=HI,
                              preferred_element_type=F32) + bn_ref[...]
        og = _onehot(br_ref[...])

        @pl.when(i == 0)
        def _():
            xs_ref[...] = jnp.zeros_like(xs_ref)
            cnt_ref[...] = jnp.zeros_like(cnt_ref)

        xs_ref[...] += jnp.dot(og, hb, precision=HI,
                               preferred_element_type=F32)
        cnt_ref[...] += jnp.dot(og, nm, precision=HI,
                                preferred_element_type=F32)

    return pl.pallas_call(
        body,
        grid=(nb,),
        in_specs=[
            pl.BlockSpec((2, BN, h + 16), lambda i: (0, i, 0)),
            pl.BlockSpec((BN, h), lambda i: (i, 0)),
            pl.BlockSpec((BN, 1), lambda i: (i, 0)),
            pl.BlockSpec((1, BN), lambda i: (0, i)),
            pl.BlockSpec((h, h), lambda i: (0, 0)),
            pl.BlockSpec((h, h), lambda i: (0, 0)),
            pl.BlockSpec((1, h), lambda i: (0, 0)),
        ],
        out_specs=[
            pl.BlockSpec((BN, h), lambda i: (i, 0)),
            pl.BlockSpec((BN, h), lambda i: (i, 0)),
            pl.BlockSpec((NG, h), lambda i: (0, 0)),
            pl.BlockSpec((NG, 1), lambda i: (0, 0)),
        ],
        out_shape=[
            jax.ShapeDtypeStruct((npad, h), F32),
            jax.ShapeDtypeStruct((npad, h), F32),
            jax.ShapeDtypeStruct((NG, h), F32),
            jax.ShapeDtypeStruct((NG, 1), F32),
        ],
    )(sp, r, nmask, b_row, w1n, w2n, bn)


def _tc_combine4(sp, dp, r, nmask, b_row):
    npad, h = r.shape
    nb = npad // BN

    def body(sp_ref, dp_ref, r_ref, nm_ref, br_ref, xs_ref):
        i = pl.program_id(0)
        hb = _combine_h(sp_ref, dp_ref, r_ref, nm_ref[...])
        og = _onehot(br_ref[...])

        @pl.when(i == 0)
        def _():
            xs_ref[...] = jnp.zeros_like(xs_ref)

        xs_ref[...] += jnp.dot(og, hb, precision=HI,
                               preferred_element_type=F32)

    return pl.pallas_call(
        body,
        grid=(nb,),
        in_specs=[
            pl.BlockSpec((2, BN, h), lambda i: (0, i, 0)),
            pl.BlockSpec((2, BN, h + 16), lambda i: (0, i, 0)),
            pl.BlockSpec((BN, h), lambda i: (i, 0)),
            pl.BlockSpec((BN, 1), lambda i: (i, 0)),
            pl.BlockSpec((1, BN), lambda i: (0, i)),
        ],
        out_specs=pl.BlockSpec((NG, h), lambda i: (0, 0)),
        out_shape=jax.ShapeDtypeStruct((NG, h), F32),
    )(sp, dp, r, nmask, b_row)


def _tc_head(xs0, xs1, xs2, xs3, cnt0, cnt1, l1w, l1b, l2wp, l2bp):
    def body(a0, a1, a2, a3, c0, c1, w1, b1, w2, b2, o_ref):
        c0v = jnp.maximum(c0[...], 1.0)
        c1v = jnp.maximum(c1[...], 1.0)
        hcat = jnp.concatenate(
            [a0[...] / c0v, a1[...] / c0v, a2[...] / c1v, a3[...] / c1v],
            axis=1)
        t = jnp.maximum(jnp.dot(hcat, w1[...], precision=HI,
                                preferred_element_type=F32) + b1[...], 0.0)
        lg = jnp.dot(t, w2[...], precision=HI,
                     preferred_element_type=F32) + b2[...]
        colv = lax.broadcasted_iota(jnp.int32, (NG, 128), 1) < 16
        lgm = jnp.where(colv, lg, -1e30)
        m = jnp.max(lgm, axis=1, keepdims=True)
        p = jnp.where(colv, jnp.exp(lgm - m), 0.0)
        lse = jnp.log(jnp.sum(p, axis=1, keepdims=True))
        o_ref[...] = lgm - m - lse

    return pl.pallas_call(
        body,
        out_shape=jax.ShapeDtypeStruct((NG, 128), F32),
    )(xs0, xs1, xs2, xs3, cnt0, cnt1, l1w, l1b, l2wp, l2bp)


def kernel(x, edge_index, batch, c1_wr, c1_wro, c1_b, c2_wr, c2_wro, c2_b,
           c3_wr, c3_wro, c3_b, c4_wr, c4_wro, c4_b, p0_w, p1_w,
           l1_w, l1_b, l2_w, l2_b):
    n, f = x.shape
    h = c1_wr.shape[1]
    npad = ((n + BNJ - 1) // BNJ) * BNJ
    pad = npad - n

    xp = jnp.pad(x, ((0, pad), (0, 0)))
    bf = jnp.pad(batch.astype(F32), (0, pad), constant_values=1e9)
    b_row = bf.reshape(1, npad)
    b_col = bf.reshape(npad, 1)
    eb = 125
    src2d = edge_index[0].reshape(-1, eb)
    dst2d = edge_index[1].reshape(-1, eb)
    zeros80 = jnp.zeros((npad, h + 16), F32)
    zeros64 = zeros80[:, :h]

    y1, r1 = _tc_pre(xp, c1_wr, c1_wro, c1_b.reshape(1, h))
    s1p = _edge_rowagg(y1, src2d, dst2d, zeros80)
    y2, r2, xs0, cnt0 = _tc_combine1(s1p, r1, b_row, c2_wr, c2_wro,
                                     c2_b.reshape(1, h))
    s2p = _edge_rowagg(y2, src2d, dst2d, zeros64)
    h2, z, xs1 = _tc_combine2(s2p, s1p, r2, b_row, p0_w.reshape(1, h))
    rank = _tc_rank(z, z.reshape(1, npad), b_col, b_row)
    y3, r3, nmask = _tc_topk(h2, z, rank, cnt0.reshape(1, NG), b_col,
                             c3_wr, c3_wro, c3_b.reshape(1, h))
    s3p = _edge_rowagg(y3, src2d, dst2d, zeros80)
    y4, r4, xs2, cnt1 = _tc_combine3(s3p, r3, nmask, b_row,
                                     c4_wr, c4_wro, c4_b.reshape(1, h))
    s4p = _edge_rowagg(y4, src2d, dst2d, zeros64)
    xs3 = _tc_combine4(s4p, s3p, r4, nmask, b_row)

    c = l2_w.shape[1]
    l2wp = jnp.pad(l2_w, ((0, 0), (0, 128 - c)))
    l2bp = jnp.pad(l2_b.reshape(1, c), ((0, 0), (0, 128 - c)))
    out = _tc_head(xs0, xs1, xs2, xs3, cnt0, cnt1, l1_w,
                   l1_b.reshape(1, h), l2wp, l2bp)
    return out[:, :c]
